# Initial kernel scaffold; baseline (speedup 1.0000x reference)
import jax
import jax.numpy as jnp
from jax.experimental import pallas as pl


def kernel(x, node_ts, nbr_ts, W1, b1, gamma1, beta1, W2, b2, gamma2, beta2, nbr_idx):
    raise NotImplementedError("write your pallas kernel here")


# speedup vs baseline: 4.8271x; 4.8271x over previous
import functools

import jax
import jax.numpy as jnp
from jax import lax
from jax.experimental import pallas as pl
from jax.experimental.pallas import tpu as pltpu
from jax.experimental.pallas import tpu_sc as plsc

N = 10000
DEG = 16
D = 128

NW = 32
NPAD = 10240
NODES_PER_W = NPAD // NW
CHN = 16
NCH_A = NODES_PER_W // CHN

SB = NPAD * DEG
PREAL = SB + NPAD
KC = 256
CH_C = 22
PW = KC * CH_C
PPAD = NW * PW

PADTS = 3.0e38
EPS = 1e-5

_f32 = jnp.float32
_i32 = jnp.int32


def _sc_mesh():
    return plsc.VectorSubcoreMesh(core_axis_name="c", subcore_axis_name="s")


_SC_PARAMS = pltpu.CompilerParams(needs_layout_passes=False)


def _wid():
    return lax.axis_index("s") * 2 + lax.axis_index("c")


def _sort_gather_body(nbt_hbm, nbi_hbm, x_hbm, child_hbm,
                      tsb, idb, sid0, sid1, rows, sem):
    base = _wid() * NODES_PER_W

    def chunk(c, carry):
        nb = base + c * CHN
        pltpu.sync_copy(nbt_hbm.at[pl.ds(nb, CHN)], tsb)
        pltpu.sync_copy(nbi_hbm.at[pl.ds(nb, CHN)], idb)
        for n in range(CHN):
            _, si = plsc.sort_key_val(tsb[n], idb[n])
            tgt = sid0 if n < 8 else sid1
            tgt[pl.ds((n % 8) * DEG, DEG)] = si
        pltpu.async_copy(x_hbm.at[sid0], rows.at[pl.ds(0, 8 * DEG)], sem).wait()
        pltpu.async_copy(x_hbm.at[sid1], rows.at[pl.ds(8 * DEG, 8 * DEG)], sem).wait()
        pltpu.sync_copy(rows, child_hbm.at[pl.ds(nb * DEG, CHN * DEG)])
        return carry

    lax.fori_loop(0, NCH_A, chunk, 0)


def _sort_gather(nbtp, nbip, xp):
    f = pl.kernel(
        _sort_gather_body,
        out_type=jax.ShapeDtypeStruct((NPAD * DEG, D), _f32),
        mesh=_sc_mesh(),
        scratch_types=[
            pltpu.VMEM((CHN, DEG), _f32),
            pltpu.VMEM((CHN, DEG), _i32),
            pltpu.VMEM((8 * DEG,), _i32),
            pltpu.VMEM((8 * DEG,), _i32),
            pltpu.VMEM((CHN * DEG, D), _f32),
            pltpu.SemaphoreType.DMA,
        ],
        compiler_params=_SC_PARAMS,
    )
    return f(nbtp, nbip, xp)


def _table_body(child_ref, x_ref, out_ref, *, bn):
    c3 = child_ref[...].reshape(bn, DEG, D)
    acc = x_ref[...]
    out_ref[0] = acc
    for k in range(1, DEG + 1):
        acc = acc + c3[:, k - 1, :]
        out_ref[k] = acc * (1.0 / (k + 1))


def _build_table(child, xp):
    bn = 128
    grid = (NPAD // bn,)
    return pl.pallas_call(
        functools.partial(_table_body, bn=bn),
        grid=grid,
        in_specs=[
            pl.BlockSpec((bn * DEG, D), lambda b: (b, 0)),
            pl.BlockSpec((bn, D), lambda b: (b, 0)),
        ],
        out_specs=pl.BlockSpec((DEG + 1, bn, D), lambda b: (0, b, 0)),
        out_shape=jax.ShapeDtypeStruct((DEG + 1, NPAD, D), _f32),
    )(child, xp)


def _rank_gather_body(pu_hbm, pt_hbm, nbt8_hbm, table_hbm, agg1_hbm,
                      uv0, uv1, ub0, ub1, tv, tsrows, ix0, ix1, outr, sem):
    base = _wid() * PW

    def chunk(c, carry):
        pb = base + c * KC
        pltpu.sync_copy(pu_hbm.at[pl.ds(pb, 128)], uv0)
        pltpu.sync_copy(pu_hbm.at[pl.ds(pb + 128, 128)], uv1)
        pltpu.sync_copy(pt_hbm.at[pl.ds(pb, KC)], tv)
        for g in range(8):
            ub0[pl.ds(g * 16, 16)] = (
                uv0[pl.ds(g * 16, 16)] >> 3).astype(_i32)
            ub1[pl.ds(g * 16, 16)] = (
                uv1[pl.ds(g * 16, 16)] >> 3).astype(_i32)
        pltpu.async_copy(nbt8_hbm.at[ub0], tsrows.at[pl.ds(0, 128)], sem).wait()
        pltpu.async_copy(nbt8_hbm.at[ub1], tsrows.at[pl.ds(128, 128)], sem).wait()
        for g in range(KC // 16):
            uref = uv0 if g < 8 else uv1
            goff = (g % 8) * 16
            u16 = uref[pl.ds(goff, 16)]
            t16 = tv[pl.ds(g * 16, 16)]
            rows16 = lax.iota(_i32, 16) + g * 16
            colbase = (u16 & 7) * DEG
            r = jnp.zeros((16,), _i32)
            for j in range(DEG):
                col = plsc.load_gather(tsrows, [rows16, colbase + j])
                r = r + (col <= t16).astype(_i32)
            iref = ix0 if g < 8 else ix1
            iref[pl.ds(goff, 16)] = r * NPAD + u16
        pltpu.async_copy(table_hbm.at[ix0], outr.at[pl.ds(0, 128)], sem).wait()
        pltpu.async_copy(table_hbm.at[ix1], outr.at[pl.ds(128, 128)], sem).wait()
        pltpu.sync_copy(outr, agg1_hbm.at[pl.ds(pb, KC)])
        return carry

    lax.fori_loop(0, CH_C, chunk, 0)


def _rank_gather(pair_u, pair_t, nbt8, table_flat):
    f = pl.kernel(
        _rank_gather_body,
        out_type=jax.ShapeDtypeStruct((PPAD, D), _f32),
        mesh=_sc_mesh(),
        scratch_types=[
            pltpu.VMEM((128,), _i32),
            pltpu.VMEM((128,), _i32),
            pltpu.VMEM((128,), _i32),
            pltpu.VMEM((128,), _i32),
            pltpu.VMEM((KC,), _f32),
            pltpu.VMEM((KC, D), _f32),
            pltpu.VMEM((128,), _i32),
            pltpu.VMEM((128,), _i32),
            pltpu.VMEM((KC, D), _f32),
            pltpu.SemaphoreType.DMA,
        ],
        compiler_params=_SC_PARAMS,
    )
    return f(pair_u, pair_t, nbt8, table_flat)


def _mm1_body(agg_ref, w1_ref, b1_ref, pw_ref, h_ref, st_ref, acc):
    step = pl.program_id(0)

    @pl.when(step == 0)
    def _():
        acc[...] = jnp.zeros_like(acc)

    a = agg_ref[...]
    h = jnp.dot(a, w1_ref[...], preferred_element_type=_f32) + b1_ref[...]
    h_ref[...] = h
    w = pw_ref[...]
    hw = h * w
    s1 = jnp.sum(hw, axis=0, keepdims=True)
    s2 = jnp.sum(h * hw, axis=0, keepdims=True)
    d = jnp.sum(w)
    acc[0:1] = acc[0:1] + s1
    acc[1:2] = acc[1:2] + s2
    acc[2:3] = acc[2:3] + jnp.full((1, D), d, _f32)
    st_ref[...] = acc[...]


def _mm1(agg1, W1, b1, pair_w):
    bp = 1024
    grid = (PPAD // bp,)
    return pl.pallas_call(
        _mm1_body,
        grid=grid,
        in_specs=[
            pl.BlockSpec((bp, D), lambda b: (b, 0)),
            pl.BlockSpec((D, D), lambda b: (0, 0)),
            pl.BlockSpec((1, D), lambda b: (0, 0)),
            pl.BlockSpec((bp, 1), lambda b: (b, 0)),
        ],
        out_specs=[
            pl.BlockSpec((bp, D), lambda b: (b, 0)),
            pl.BlockSpec((8, D), lambda b: (0, 0)),
        ],
        out_shape=[
            jax.ShapeDtypeStruct((PPAD, D), _f32),
            jax.ShapeDtypeStruct((8, D), _f32),
        ],
        scratch_shapes=[pltpu.VMEM((8, D), _f32)],
    )(agg1, W1, b1, pair_w)


def _layer2_body(hn_ref, hs_ref, m2_ref, st1_ref, g1_ref, be1_ref,
                 w2_ref, b2_ref, h2_ref, st2_ref, acc, *, bn):
    step = pl.program_id(0)

    @pl.when(step == 0)
    def _():
        acc[...] = jnp.zeros_like(acc)

    den = jnp.maximum(st1_ref[2:3], 1.0)
    mean = st1_ref[0:1] / den
    var = st1_ref[1:2] / den - mean * mean
    sc = g1_ref[...] * lax.rsqrt(var + EPS)
    sh = be1_ref[...] - mean * sc
    sc3 = sc.reshape(1, 1, D)
    sh3 = sh.reshape(1, 1, D)

    hn3 = hn_ref[...].reshape(bn, DEG, D)
    m2 = m2_ref[...]
    num = jax.nn.relu(hs_ref[...] * sc + sh)
    for k in range(DEG):
        h1k = jax.nn.relu(hn3[:, k, :] * sc3[0] + sh3[0])
        num = num + h1k * m2[:, k:k + 1]
    degs = jnp.sum(m2, axis=1, keepdims=True) + 1.0
    agg2 = num / degs
    h2p = jnp.dot(agg2, w2_ref[...], preferred_element_type=_f32) + b2_ref[...]
    h2_ref[...] = h2p

    rid = lax.broadcasted_iota(_i32, (bn, 1), 0) + step * bn
    w = (rid < N).astype(_f32)
    hw = h2p * w
    acc[0:1] = acc[0:1] + jnp.sum(hw, axis=0, keepdims=True)
    acc[1:2] = acc[1:2] + jnp.sum(h2p * hw, axis=0, keepdims=True)
    acc[2:3] = acc[2:3] + jnp.full((1, D), jnp.sum(w), _f32)
    st2_ref[...] = acc[...]


def _layer2(h1pre, mask2, stats1, gamma1, beta1, W2, b2):
    bn = 128
    grid = (NPAD // bn,)
    sb_blk = SB // bn
    return pl.pallas_call(
        functools.partial(_layer2_body, bn=bn),
        grid=grid,
        in_specs=[
            pl.BlockSpec((bn * DEG, D), lambda b: (b, 0)),
            pl.BlockSpec((bn, D), lambda b: (sb_blk + b, 0)),
            pl.BlockSpec((bn, DEG), lambda b: (b, 0)),
            pl.BlockSpec((8, D), lambda b: (0, 0)),
            pl.BlockSpec((1, D), lambda b: (0, 0)),
            pl.BlockSpec((1, D), lambda b: (0, 0)),
            pl.BlockSpec((D, D), lambda b: (0, 0)),
            pl.BlockSpec((1, D), lambda b: (0, 0)),
        ],
        out_specs=[
            pl.BlockSpec((bn, D), lambda b: (b, 0)),
            pl.BlockSpec((8, D), lambda b: (0, 0)),
        ],
        out_shape=[
            jax.ShapeDtypeStruct((NPAD, D), _f32),
            jax.ShapeDtypeStruct((8, D), _f32),
        ],
        scratch_shapes=[pltpu.VMEM((8, D), _f32)],
    )(h1pre, h1pre, mask2, stats1, gamma1, beta1, W2, b2)


def _final_body(h2_ref, st2_ref, g2_ref, be2_ref, out_ref):
    den = jnp.maximum(st2_ref[2:3], 1.0)
    mean = st2_ref[0:1] / den
    var = st2_ref[1:2] / den - mean * mean
    sc = g2_ref[...] * lax.rsqrt(var + EPS)
    sh = be2_ref[...] - mean * sc
    out_ref[...] = jax.nn.relu(h2_ref[...] * sc + sh)


def _final(h2pre, stats2, gamma2, beta2):
    bn = 512
    grid = (NPAD // bn,)
    return pl.pallas_call(
        _final_body,
        grid=grid,
        in_specs=[
            pl.BlockSpec((bn, D), lambda b: (b, 0)),
            pl.BlockSpec((8, D), lambda b: (0, 0)),
            pl.BlockSpec((1, D), lambda b: (0, 0)),
            pl.BlockSpec((1, D), lambda b: (0, 0)),
        ],
        out_specs=pl.BlockSpec((bn, D), lambda b: (b, 0)),
        out_shape=jax.ShapeDtypeStruct((NPAD, D), _f32),
    )(h2pre, stats2, gamma2, beta2)


@jax.jit
def _run(x, node_ts, nbr_ts, W1, b1, gamma1, beta1, W2, b2, gamma2, beta2,
         nbr_idx):
    xp = jnp.zeros((NPAD, D), _f32).at[:N].set(x)
    ntp = jnp.zeros((NPAD,), _f32).at[:N].set(node_ts)
    nbtp = jnp.full((NPAD, DEG), PADTS, _f32).at[:N].set(nbr_ts)
    nbip = jnp.zeros((NPAD, DEG), _i32).at[:N].set(nbr_idx.astype(_i32))
    mask2 = (nbtp <= ntp[:, None]).astype(_f32)
    wself = jnp.zeros((NPAD,), _f32).at[:N].set(1.0)
    zpad_i = jnp.zeros((PPAD - PREAL,), _i32)
    zpad_f = jnp.zeros((PPAD - PREAL,), _f32)
    pair_u = jnp.concatenate(
        [nbip.reshape(-1), jnp.arange(NPAD, dtype=_i32), zpad_i])
    pair_t = jnp.concatenate([nbtp.reshape(-1), ntp, zpad_f])
    pair_w = jnp.concatenate([mask2.reshape(-1), wself, zpad_f])

    child = _sort_gather(nbtp, nbip, xp)
    table = _build_table(child, xp)
    table_flat = table.reshape((DEG + 1) * NPAD, D)
    agg1 = _rank_gather(pair_u, pair_t, nbtp.reshape(NPAD // 8, 8 * DEG),
                        table_flat)
    h1pre, stats1 = _mm1(agg1, W1, b1.reshape(1, D), pair_w.reshape(PPAD, 1))
    h2pre, stats2 = _layer2(h1pre, mask2, stats1,
                            gamma1.reshape(1, D), beta1.reshape(1, D),
                            W2, b2.reshape(1, D))
    h2 = _final(h2pre, stats2, gamma2.reshape(1, D), beta2.reshape(1, D))
    return h2[:N]


def kernel(x, node_ts, nbr_ts, W1, b1, gamma1, beta1, W2, b2, gamma2, beta2,
           nbr_idx):
    return _run(x, node_ts, nbr_ts, W1, b1, gamma1, beta1, W2, b2,
                gamma2, beta2, nbr_idx)


# speedup vs baseline: 5.1419x; 1.0652x over previous
import functools

import jax
import jax.numpy as jnp
from jax import lax
from jax.experimental import pallas as pl
from jax.experimental.pallas import tpu as pltpu
from jax.experimental.pallas import tpu_sc as plsc

N = 10000
DEG = 16
D = 128

NW = 32
NPAD = 10240
NODES_PER_W = NPAD // NW
CHN = 16
NCH_A = NODES_PER_W // CHN

SB = NPAD * DEG
PREAL = SB + NPAD
KC = 128
CH_C = 44
KC2 = 256
CH_C2 = 22
PW = KC * CH_C
PPAD = NW * PW

PADTS = 3.0e38
EPS = 1e-5

_f32 = jnp.float32
_i32 = jnp.int32


def _sc_mesh():
    return plsc.VectorSubcoreMesh(core_axis_name="c", subcore_axis_name="s")


_SC_PARAMS = pltpu.CompilerParams(needs_layout_passes=False)


def _wid():
    return lax.axis_index("s") * 2 + lax.axis_index("c")


def _sga_sorts(tid, sid_pair):
    for n in range(CHN):
        ts = plsc.bitcast(tid[n, pl.ds(0, DEG)], _f32)
        ids = tid[n, pl.ds(DEG, DEG)]
        _, si = plsc.sort_key_val(ts, ids)
        sid_pair[n // 8][pl.ds((n % 8) * DEG, DEG)] = si


def _sort_gather_body(tid_hbm, x_hbm, child_hbm,
                      tid0, tid1, tid2, tid3, sa0, sb0, sa1, sb1,
                      rows0, rows1,
                      st0, st1, st2, st3, sg0, sg1, sw0, sw1):
    base = _wid() * NODES_PER_W
    tid = [tid0, tid1, tid2, tid3]
    st = [st0, st1, st2, st3]
    sid = [(sa0, sb0), (sa1, sb1)]
    rows = [rows0, rows1]
    sg = [sg0, sg1]
    sw = [sw0, sw1]

    def tid_copy(c, s):
        pltpu.async_copy(tid_hbm.at[pl.ds(base + c * CHN, CHN)], tid[s], st[s])

    def tid_wait(s):
        pltpu.make_async_copy(tid_hbm.at[pl.ds(0, CHN)], tid[s], st[s]).wait()

    def x_gather(b):
        pltpu.async_copy(x_hbm.at[sid[b][0]], rows[b].at[pl.ds(0, 128)], sg[b])
        pltpu.async_copy(
            x_hbm.at[sid[b][1]], rows[b].at[pl.ds(128, 128)], sg[b])

    def x_gather_wait(b):
        for h in range(2):
            pltpu.make_async_copy(
                x_hbm.at[pl.ds(0, 128)], rows[b].at[pl.ds(h * 128, 128)],
                sg[b]).wait()

    tid_copy(0, 0)
    tid_copy(1, 1)
    tid_wait(0)
    _sga_sorts(tid0, sid[0])
    x_gather(0)

    def outer(g, carry):
        for j in range(4):
            c = g * 4 + j
            s1, s2 = (j + 1) % 4, (j + 2) % 4
            b, b1 = j % 2, (j + 1) % 2

            @pl.when(c + 2 < NCH_A)
            def _():
                tid_copy(c + 2, s2)

            @pl.when(c + 1 < NCH_A)
            def _():
                tid_wait(s1)
                _sga_sorts(tid[s1], sid[b1])

                @pl.when(c >= 1)
                def _():
                    pltpu.make_async_copy(
                        rows[b1], child_hbm.at[pl.ds(0, CHN * DEG)],
                        sw[b1]).wait()

                x_gather(b1)

            x_gather_wait(b)
            nb = base + c * CHN
            pltpu.async_copy(
                rows[b], child_hbm.at[pl.ds(nb * DEG, CHN * DEG)], sw[b])
        return carry

    lax.fori_loop(0, NCH_A // 4, outer, 0)
    for b in range(2):
        pltpu.make_async_copy(
            rows[b], child_hbm.at[pl.ds(0, CHN * DEG)], sw[b]).wait()


def _sort_gather(tid_packed, xp):
    f = pl.kernel(
        _sort_gather_body,
        out_type=jax.ShapeDtypeStruct((NPAD * DEG, D), _f32),
        mesh=_sc_mesh(),
        scratch_types=(
            [pltpu.VMEM((CHN, 2 * DEG), _i32) for _ in range(4)]
            + [pltpu.VMEM((128,), _i32) for _ in range(4)]
            + [pltpu.VMEM((CHN * DEG, D), _f32) for _ in range(2)]
            + [pltpu.SemaphoreType.DMA for _ in range(8)]
        ),
        compiler_params=_SC_PARAMS,
    )
    return f(tid_packed, xp)


def _table_body(child_ref, x_ref, out_ref, *, bn):
    c3 = child_ref[...].reshape(bn, DEG, D)
    acc = x_ref[...]
    out_ref[0] = acc
    for k in range(1, DEG + 1):
        acc = acc + c3[:, k - 1, :]
        out_ref[k] = acc * (1.0 / (k + 1))


def _build_table(child, xp):
    bn = 128
    grid = (NPAD // bn,)
    return pl.pallas_call(
        functools.partial(_table_body, bn=bn),
        grid=grid,
        in_specs=[
            pl.BlockSpec((bn * DEG, D), lambda b: (b, 0)),
            pl.BlockSpec((bn, D), lambda b: (b, 0)),
        ],
        out_specs=pl.BlockSpec((DEG + 1, bn, D), lambda b: (0, b, 0)),
        out_shape=jax.ShapeDtypeStruct((DEG + 1, NPAD, D), _f32),
    )(child, xp)


def _rgb_build_ub(uts, ub):
    for g in range(8):
        ub[pl.ds(g * 16, 16)] = uts[pl.ds(g * 16, 16)] >> 3


def _rgb_ranks(uts, tsr, ixm, cbase):
    for g in range(8):
        u16 = uts[pl.ds(g * 16, 16)]
        t16 = plsc.bitcast(uts[pl.ds(128 + g * 16, 16)], _f32)
        rows16 = lax.iota(_i32, 16) + g * 16
        colbase = (u16 & 7) * DEG
        r = jnp.zeros((16,), _i32)
        for j in range(DEG):
            col = plsc.load_gather(tsr, [rows16, colbase + j])
            r = r + (col <= t16).astype(_i32)
        ixm[pl.ds(cbase + g * 16, 16)] = r * NPAD + u16


def _rank_gather_body(uts_hbm, nbt8_hbm, table_hbm, agg1_hbm,
                      uts0, uts1, uts2, uts3, ub0, ub1, tsr0, tsr1, ixm,
                      outr0, outr1,
                      su0, su1, su2, su3, sts0, sts1, sg0, sg1, sw0, sw1):
    w = _wid()
    ubase = w * CH_C
    pbase = w * PW
    uts = [uts0, uts1, uts2, uts3]
    su = [su0, su1, su2, su3]
    ub = [ub0, ub1]
    tsr = [tsr0, tsr1]
    sts = [sts0, sts1]
    outr = [outr0, outr1]
    sg = [sg0, sg1]
    sw = [sw0, sw1]

    def uts_copy(c, s):
        pltpu.async_copy(uts_hbm.at[ubase + c], uts[s], su[s])

    def uts_wait(s):
        pltpu.make_async_copy(uts_hbm.at[0], uts[s], su[s]).wait()

    def ts_gather(c, s, b):
        _rgb_build_ub(uts[s], ub[b])
        pltpu.async_copy(nbt8_hbm.at[ub[b]], tsr[b], sts[b])

    uts_copy(0, 0)
    uts_copy(1, 1)
    uts_wait(0)
    ts_gather(0, 0, 0)

    def p1_outer(g, carry):
        for j in range(4):
            c = g * 4 + j
            s1, s2 = (j + 1) % 4, (j + 2) % 4
            b, b1 = j % 2, (j + 1) % 2

            @pl.when(c + 2 < CH_C)
            def _():
                uts_copy(c + 2, s2)

            @pl.when(c + 1 < CH_C)
            def _():
                uts_wait(s1)
                ts_gather(c + 1, s1, b1)

            pltpu.make_async_copy(
                nbt8_hbm.at[pl.ds(0, KC)], tsr[b], sts[b]).wait()
            _rgb_ranks(uts[j], tsr[b], ixm, c * KC)
        return carry

    lax.fori_loop(0, CH_C // 4, p1_outer, 0)

    def tab_gather(c2, b):
        for h in range(2):
            pltpu.async_copy(
                table_hbm.at[ixm.at[pl.ds(c2 * KC2 + h * 128, 128)]],
                outr[b].at[pl.ds(h * 128, 128)], sg[b])

    def tab_gather_wait(b):
        for h in range(2):
            pltpu.make_async_copy(
                table_hbm.at[pl.ds(0, 128)], outr[b].at[pl.ds(h * 128, 128)],
                sg[b]).wait()

    tab_gather(0, 0)

    def p2_outer(g, carry):
        for j in range(2):
            c2 = g * 2 + j
            b, b1 = j, (j + 1) % 2

            @pl.when(c2 + 1 < CH_C2)
            def _():
                @pl.when(c2 >= 1)
                def _():
                    pltpu.make_async_copy(
                        outr[b1], agg1_hbm.at[pl.ds(0, KC2)], sw[b1]).wait()

                tab_gather(c2 + 1, b1)

            tab_gather_wait(b)
            pltpu.async_copy(
                outr[b], agg1_hbm.at[pl.ds(pbase + c2 * KC2, KC2)], sw[b])
        return carry

    lax.fori_loop(0, CH_C2 // 2, p2_outer, 0)
    for b in range(2):
        pltpu.make_async_copy(
            outr[b], agg1_hbm.at[pl.ds(0, KC2)], sw[b]).wait()


def _rank_gather(uts_packed, nbt8, table_flat):
    f = pl.kernel(
        _rank_gather_body,
        out_type=jax.ShapeDtypeStruct((PPAD, D), _f32),
        mesh=_sc_mesh(),
        scratch_types=(
            [pltpu.VMEM((2 * KC,), _i32) for _ in range(4)]
            + [pltpu.VMEM((KC,), _i32) for _ in range(2)]
            + [pltpu.VMEM((KC, KC), _f32) for _ in range(2)]
            + [pltpu.VMEM((PW,), _i32)]
            + [pltpu.VMEM((KC2, D), _f32) for _ in range(2)]
            + [pltpu.SemaphoreType.DMA for _ in range(10)]
        ),
        compiler_params=_SC_PARAMS,
    )
    return f(uts_packed, nbt8, table_flat)


def _mm1_body(agg_ref, w1_ref, b1_ref, pw_ref, h_ref, st_ref, acc):
    step = pl.program_id(0)

    @pl.when(step == 0)
    def _():
        acc[...] = jnp.zeros_like(acc)

    a = agg_ref[...]
    h = jnp.dot(a, w1_ref[...], preferred_element_type=_f32) + b1_ref[...]
    h_ref[...] = h
    w = pw_ref[...]
    hw = h * w
    s1 = jnp.sum(hw, axis=0, keepdims=True)
    s2 = jnp.sum(h * hw, axis=0, keepdims=True)
    d = jnp.sum(w)
    acc[0:1] = acc[0:1] + s1
    acc[1:2] = acc[1:2] + s2
    acc[2:3] = acc[2:3] + jnp.full((1, D), d, _f32)
    st_ref[...] = acc[...]


def _mm1(agg1, W1, b1, pair_w):
    bp = 1024
    grid = (PPAD // bp,)
    return pl.pallas_call(
        _mm1_body,
        grid=grid,
        in_specs=[
            pl.BlockSpec((bp, D), lambda b: (b, 0)),
            pl.BlockSpec((D, D), lambda b: (0, 0)),
            pl.BlockSpec((1, D), lambda b: (0, 0)),
            pl.BlockSpec((bp, 1), lambda b: (b, 0)),
        ],
        out_specs=[
            pl.BlockSpec((bp, D), lambda b: (b, 0)),
            pl.BlockSpec((8, D), lambda b: (0, 0)),
        ],
        out_shape=[
            jax.ShapeDtypeStruct((PPAD, D), _f32),
            jax.ShapeDtypeStruct((8, D), _f32),
        ],
        scratch_shapes=[pltpu.VMEM((8, D), _f32)],
    )(agg1, W1, b1, pair_w)


def _layer2_body(hn_ref, hs_ref, m2_ref, st1_ref, g1_ref, be1_ref,
                 w2_ref, b2_ref, h2_ref, st2_ref, acc, *, bn):
    step = pl.program_id(0)

    @pl.when(step == 0)
    def _():
        acc[...] = jnp.zeros_like(acc)

    den = jnp.maximum(st1_ref[2:3], 1.0)
    mean = st1_ref[0:1] / den
    var = st1_ref[1:2] / den - mean * mean
    sc = g1_ref[...] * lax.rsqrt(var + EPS)
    sh = be1_ref[...] - mean * sc
    sc3 = sc.reshape(1, 1, D)
    sh3 = sh.reshape(1, 1, D)

    hn3 = hn_ref[...].reshape(bn, DEG, D)
    m2 = m2_ref[...]
    num = jax.nn.relu(hs_ref[...] * sc + sh)
    for k in range(DEG):
        h1k = jax.nn.relu(hn3[:, k, :] * sc3[0] + sh3[0])
        num = num + h1k * m2[:, k:k + 1]
    degs = jnp.sum(m2, axis=1, keepdims=True) + 1.0
    agg2 = num / degs
    h2p = jnp.dot(agg2, w2_ref[...], preferred_element_type=_f32) + b2_ref[...]
    h2_ref[...] = h2p

    rid = lax.broadcasted_iota(_i32, (bn, 1), 0) + step * bn
    w = (rid < N).astype(_f32)
    hw = h2p * w
    acc[0:1] = acc[0:1] + jnp.sum(hw, axis=0, keepdims=True)
    acc[1:2] = acc[1:2] + jnp.sum(h2p * hw, axis=0, keepdims=True)
    acc[2:3] = acc[2:3] + jnp.full((1, D), jnp.sum(w), _f32)
    st2_ref[...] = acc[...]


def _layer2(h1pre, mask2, stats1, gamma1, beta1, W2, b2):
    bn = 128
    grid = (NPAD // bn,)
    sb_blk = SB // bn
    return pl.pallas_call(
        functools.partial(_layer2_body, bn=bn),
        grid=grid,
        in_specs=[
            pl.BlockSpec((bn * DEG, D), lambda b: (b, 0)),
            pl.BlockSpec((bn, D), lambda b: (sb_blk + b, 0)),
            pl.BlockSpec((bn, DEG), lambda b: (b, 0)),
            pl.BlockSpec((8, D), lambda b: (0, 0)),
            pl.BlockSpec((1, D), lambda b: (0, 0)),
            pl.BlockSpec((1, D), lambda b: (0, 0)),
            pl.BlockSpec((D, D), lambda b: (0, 0)),
            pl.BlockSpec((1, D), lambda b: (0, 0)),
        ],
        out_specs=[
            pl.BlockSpec((bn, D), lambda b: (b, 0)),
            pl.BlockSpec((8, D), lambda b: (0, 0)),
        ],
        out_shape=[
            jax.ShapeDtypeStruct((NPAD, D), _f32),
            jax.ShapeDtypeStruct((8, D), _f32),
        ],
        scratch_shapes=[pltpu.VMEM((8, D), _f32)],
    )(h1pre, h1pre, mask2, stats1, gamma1, beta1, W2, b2)


def _final_body(h2_ref, st2_ref, g2_ref, be2_ref, out_ref):
    den = jnp.maximum(st2_ref[2:3], 1.0)
    mean = st2_ref[0:1] / den
    var = st2_ref[1:2] / den - mean * mean
    sc = g2_ref[...] * lax.rsqrt(var + EPS)
    sh = be2_ref[...] - mean * sc
    out_ref[...] = jax.nn.relu(h2_ref[...] * sc + sh)


def _final(h2pre, stats2, gamma2, beta2):
    bn = 512
    grid = (NPAD // bn,)
    return pl.pallas_call(
        _final_body,
        grid=grid,
        in_specs=[
            pl.BlockSpec((bn, D), lambda b: (b, 0)),
            pl.BlockSpec((8, D), lambda b: (0, 0)),
            pl.BlockSpec((1, D), lambda b: (0, 0)),
            pl.BlockSpec((1, D), lambda b: (0, 0)),
        ],
        out_specs=pl.BlockSpec((bn, D), lambda b: (b, 0)),
        out_shape=jax.ShapeDtypeStruct((NPAD, D), _f32),
    )(h2pre, stats2, gamma2, beta2)


@jax.jit
def _run(x, node_ts, nbr_ts, W1, b1, gamma1, beta1, W2, b2, gamma2, beta2,
         nbr_idx):
    xp = jnp.zeros((NPAD, D), _f32).at[:N].set(x)
    ntp = jnp.zeros((NPAD,), _f32).at[:N].set(node_ts)
    nbtp = jnp.full((NPAD, DEG), PADTS, _f32).at[:N].set(nbr_ts)
    nbip = jnp.zeros((NPAD, DEG), _i32).at[:N].set(nbr_idx.astype(_i32))
    mask2 = (nbtp <= ntp[:, None]).astype(_f32)
    wself = jnp.zeros((NPAD,), _f32).at[:N].set(1.0)
    zpad_i = jnp.zeros((PPAD - PREAL,), _i32)
    zpad_f = jnp.zeros((PPAD - PREAL,), _f32)
    pair_u = jnp.concatenate(
        [nbip.reshape(-1), jnp.arange(NPAD, dtype=_i32), zpad_i])
    pair_t = jnp.concatenate([nbtp.reshape(-1), ntp, zpad_f])
    pair_w = jnp.concatenate([mask2.reshape(-1), wself, zpad_f])

    tid_packed = jnp.concatenate(
        [lax.bitcast_convert_type(nbtp, _i32), nbip], axis=1)
    u_r = pair_u.reshape(NW, CH_C, KC)
    t_r = lax.bitcast_convert_type(pair_t, _i32).reshape(NW, CH_C, KC)
    uts_packed = jnp.concatenate([u_r, t_r], axis=2).reshape(NW * CH_C, 2 * KC)

    child = _sort_gather(tid_packed, xp)
    table = _build_table(child, xp)
    table_flat = table.reshape((DEG + 1) * NPAD, D)
    agg1 = _rank_gather(uts_packed, nbtp.reshape(NPAD // 8, 8 * DEG),
                        table_flat)
    h1pre, stats1 = _mm1(agg1, W1, b1.reshape(1, D), pair_w.reshape(PPAD, 1))
    h2pre, stats2 = _layer2(h1pre, mask2, stats1,
                            gamma1.reshape(1, D), beta1.reshape(1, D),
                            W2, b2.reshape(1, D))
    h2 = _final(h2pre, stats2, gamma2.reshape(1, D), beta2.reshape(1, D))
    return h2[:N]


def kernel(x, node_ts, nbr_ts, W1, b1, gamma1, beta1, W2, b2, gamma2, beta2,
           nbr_idx):
    return _run(x, node_ts, nbr_ts, W1, b1, gamma1, beta1, W2, b2,
                gamma2, beta2, nbr_idx)


# speedup vs baseline: 6.5095x; 1.2660x over previous
import functools

import jax
import jax.numpy as jnp
from jax import lax
from jax.experimental import pallas as pl
from jax.experimental.pallas import tpu as pltpu
from jax.experimental.pallas import tpu_sc as plsc

N = 10000
DEG = 16
D = 128

NW = 32
NPAD = 10240
NODES_PER_W = NPAD // NW
CHN = 8
NCH_A = NODES_PER_W // CHN

SB = NPAD * DEG
PREAL = SB + NPAD
KC = 128
CH_C = 44
KC2 = 256
CH_C2 = 22
PW = KC * CH_C
PPAD = NW * PW

PADTS = 3.0e38
EPS = 1e-5

_f32 = jnp.float32
_i32 = jnp.int32


def _sc_mesh():
    return plsc.VectorSubcoreMesh(core_axis_name="c", subcore_axis_name="s")


_SC_PARAMS = pltpu.CompilerParams(needs_layout_passes=False)


def _wid():
    return lax.axis_index("s") * 2 + lax.axis_index("c")


def _sga_sorts(tid, sidx, uidx):
    for n in range(CHN):
        ts = plsc.bitcast(tid[n, pl.ds(0, DEG)], _f32)
        ids = tid[n, pl.ds(DEG, DEG)]
        _, si = plsc.sort_key_val(ts, ids)
        sidx[pl.ds(n * DEG, DEG)] = si
        uidx[pl.ds(n * DEG, DEG)] = ids


def _sort_gather_body(tid_hbm, x_hbm, nbtw_hbm, child_hbm, tsg_hbm,
                      tid0, tid1, tid2, tid3, sx0, sx1, ux0, ux1,
                      rx0, rx1, rt0, rt1,
                      st0, st1, st2, st3, sgx0, sgx1, sgt0, sgt1,
                      swx0, swx1, swt0, swt1):
    base = _wid() * NODES_PER_W
    tid = [tid0, tid1, tid2, tid3]
    st = [st0, st1, st2, st3]
    sidx = [sx0, sx1]
    uidx = [ux0, ux1]
    rx = [rx0, rx1]
    rt = [rt0, rt1]
    sgx = [sgx0, sgx1]
    sgt = [sgt0, sgt1]
    swx = [swx0, swx1]
    swt = [swt0, swt1]
    NR = CHN * DEG

    def tid_copy(c, s):
        pltpu.async_copy(tid_hbm.at[pl.ds(base + c * CHN, CHN)], tid[s], st[s])

    def tid_wait(s):
        pltpu.make_async_copy(tid_hbm.at[pl.ds(0, CHN)], tid[s], st[s]).wait()

    def gathers(b):
        pltpu.async_copy(x_hbm.at[sidx[b]], rx[b], sgx[b])
        pltpu.async_copy(nbtw_hbm.at[uidx[b]], rt[b], sgt[b])

    def gathers_wait(b):
        pltpu.make_async_copy(x_hbm.at[pl.ds(0, NR)], rx[b], sgx[b]).wait()
        pltpu.make_async_copy(nbtw_hbm.at[pl.ds(0, NR)], rt[b], sgt[b]).wait()

    def writes_wait(b):
        pltpu.make_async_copy(rx[b], child_hbm.at[pl.ds(0, NR)], swx[b]).wait()
        pltpu.make_async_copy(rt[b], tsg_hbm.at[pl.ds(0, NR)], swt[b]).wait()

    tid_copy(0, 0)
    tid_copy(1, 1)
    tid_wait(0)
    _sga_sorts(tid0, sx0, ux0)
    gathers(0)

    def outer(g, carry):
        for j in range(4):
            c = g * 4 + j
            s1, s2 = (j + 1) % 4, (j + 2) % 4
            b, b1 = j % 2, (j + 1) % 2

            @pl.when(c + 2 < NCH_A)
            def _():
                tid_copy(c + 2, s2)

            @pl.when(c + 1 < NCH_A)
            def _():
                tid_wait(s1)
                _sga_sorts(tid[s1], sidx[b1], uidx[b1])

                @pl.when(c >= 1)
                def _():
                    writes_wait(b1)

                gathers(b1)

            gathers_wait(b)
            nb = base + c * CHN
            pltpu.async_copy(
                rx[b], child_hbm.at[pl.ds(nb * DEG, NR)], swx[b])
            pltpu.async_copy(
                rt[b], tsg_hbm.at[pl.ds(nb * DEG, NR)], swt[b])
        return carry

    lax.fori_loop(0, NCH_A // 4, outer, 0)
    for b in range(2):
        writes_wait(b)


def _sort_gather(tid_packed, xp, nbt_wide):
    f = pl.kernel(
        _sort_gather_body,
        out_type=(
            jax.ShapeDtypeStruct((NPAD * DEG, D), _f32),
            jax.ShapeDtypeStruct((NPAD * DEG, D), _f32),
        ),
        mesh=_sc_mesh(),
        scratch_types=(
            [pltpu.VMEM((CHN, 2 * DEG), _i32) for _ in range(4)]
            + [pltpu.VMEM((CHN * DEG,), _i32) for _ in range(4)]
            + [pltpu.VMEM((CHN * DEG, D), _f32) for _ in range(4)]
            + [pltpu.SemaphoreType.DMA for _ in range(12)]
        ),
        compiler_params=_SC_PARAMS,
    )
    return f(tid_packed, xp, nbt_wide)


def _table_body(child_ref, x_ref, tsg_ref, nbt_ref, nts_ref, nbi_ref,
                table_ref, idxn_ref, idxs_ref, *, bn):
    c3 = child_ref[...].reshape(bn, DEG, D)
    acc = x_ref[...]
    table_ref[0] = acc
    for k in range(1, DEG + 1):
        acc = acc + c3[:, k - 1, :]
        table_ref[k] = acc * (1.0 / (k + 1))
    tsg3 = tsg_ref[...].reshape(bn, DEG, D)[:, :, :DEG]
    t3 = nbt_ref[...][:, :, None]
    r = jnp.sum((tsg3 <= t3).astype(_i32), axis=2)
    idxn_ref[...] = r * NPAD + nbi_ref[...]
    rs = jnp.sum((nbt_ref[...] <= nts_ref[...]).astype(_i32), axis=1,
                 keepdims=True)
    nid = lax.broadcasted_iota(_i32, (bn, 1), 0) + pl.program_id(0) * bn
    idxs_ref[...] = rs * NPAD + nid


def _build_table(child, tsg, xp, nbtp, ntp_col, nbip):
    bn = 128
    grid = (NPAD // bn,)
    return pl.pallas_call(
        functools.partial(_table_body, bn=bn),
        grid=grid,
        in_specs=[
            pl.BlockSpec((bn * DEG, D), lambda b: (b, 0)),
            pl.BlockSpec((bn, D), lambda b: (b, 0)),
            pl.BlockSpec((bn * DEG, D), lambda b: (b, 0)),
            pl.BlockSpec((bn, DEG), lambda b: (b, 0)),
            pl.BlockSpec((bn, 1), lambda b: (b, 0)),
            pl.BlockSpec((bn, DEG), lambda b: (b, 0)),
        ],
        out_specs=[
            pl.BlockSpec((DEG + 1, bn, D), lambda b: (0, b, 0)),
            pl.BlockSpec((bn, DEG), lambda b: (b, 0)),
            pl.BlockSpec((bn, 1), lambda b: (b, 0)),
        ],
        out_shape=[
            jax.ShapeDtypeStruct((DEG + 1, NPAD, D), _f32),
            jax.ShapeDtypeStruct((NPAD, DEG), _i32),
            jax.ShapeDtypeStruct((NPAD, 1), _i32),
        ],
    )(child, xp, tsg, nbtp, ntp_col, nbip)


def _tab_gather_body(idx_hbm, table_hbm, agg1_hbm,
                     ix00, ix01, ix10, ix11, outr0, outr1,
                     si0, si1, sg0, sg1, sw0, sw1):
    base = _wid() * PW
    ix = [(ix00, ix01), (ix10, ix11)]
    si = [si0, si1]
    outr = [outr0, outr1]
    sg = [sg0, sg1]
    sw = [sw0, sw1]

    def idx_copy(c, b):
        pb = base + c * KC2
        pltpu.async_copy(idx_hbm.at[pl.ds(pb, 128)], ix[b][0], si[b])
        pltpu.async_copy(idx_hbm.at[pl.ds(pb + 128, 128)], ix[b][1], si[b])

    def idx_wait(b):
        for h in range(2):
            pltpu.make_async_copy(
                idx_hbm.at[pl.ds(0, 128)], ix[b][h], si[b]).wait()

    def tgather(b):
        for h in range(2):
            pltpu.async_copy(
                table_hbm.at[ix[b][h]], outr[b].at[pl.ds(h * 128, 128)],
                sg[b])

    def tgather_wait(b):
        for h in range(2):
            pltpu.make_async_copy(
                table_hbm.at[pl.ds(0, 128)], outr[b].at[pl.ds(h * 128, 128)],
                sg[b]).wait()

    idx_copy(0, 0)
    idx_copy(1, 1)
    idx_wait(0)
    tgather(0)

    def outer(g, carry):
        for j in range(2):
            c = g * 2 + j
            b, b1 = j, (j + 1) % 2

            @pl.when(c + 1 < CH_C2)
            def _():
                idx_wait(b1)

                @pl.when(c >= 1)
                def _():
                    pltpu.make_async_copy(
                        outr[b1], agg1_hbm.at[pl.ds(0, KC2)], sw[b1]).wait()

                tgather(b1)

            tgather_wait(b)
            pltpu.async_copy(
                outr[b], agg1_hbm.at[pl.ds(base + c * KC2, KC2)], sw[b])

            @pl.when(c + 2 < CH_C2)
            def _():
                idx_copy(c + 2, b)
        return carry

    lax.fori_loop(0, CH_C2 // 2, outer, 0)
    for b in range(2):
        pltpu.make_async_copy(
            outr[b], agg1_hbm.at[pl.ds(0, KC2)], sw[b]).wait()


def _tab_gather(idx_flat, table_flat):
    f = pl.kernel(
        _tab_gather_body,
        out_type=jax.ShapeDtypeStruct((PPAD, D), _f32),
        mesh=_sc_mesh(),
        scratch_types=(
            [pltpu.VMEM((128,), _i32) for _ in range(4)]
            + [pltpu.VMEM((KC2, D), _f32) for _ in range(2)]
            + [pltpu.SemaphoreType.DMA for _ in range(6)]
        ),
        compiler_params=_SC_PARAMS,
    )
    return f(idx_flat, table_flat)


def _mm1_body(agg_ref, w1_ref, b1_ref, pw_ref, h_ref, st_ref, acc):
    step = pl.program_id(0)

    @pl.when(step == 0)
    def _():
        acc[...] = jnp.zeros_like(acc)

    a = agg_ref[...]
    h = jnp.dot(a, w1_ref[...], preferred_element_type=_f32) + b1_ref[...]
    h_ref[...] = h
    w = pw_ref[...]
    hw = h * w
    s1 = jnp.sum(hw, axis=0, keepdims=True)
    s2 = jnp.sum(h * hw, axis=0, keepdims=True)
    d = jnp.sum(w)
    acc[0:1] = acc[0:1] + s1
    acc[1:2] = acc[1:2] + s2
    acc[2:3] = acc[2:3] + jnp.full((1, D), d, _f32)
    st_ref[...] = acc[...]


def _mm1(agg1, W1, b1, pair_w):
    bp = 1024
    grid = (PPAD // bp,)
    return pl.pallas_call(
        _mm1_body,
        grid=grid,
        in_specs=[
            pl.BlockSpec((bp, D), lambda b: (b, 0)),
            pl.BlockSpec((D, D), lambda b: (0, 0)),
            pl.BlockSpec((1, D), lambda b: (0, 0)),
            pl.BlockSpec((bp, 1), lambda b: (b, 0)),
        ],
        out_specs=[
            pl.BlockSpec((bp, D), lambda b: (b, 0)),
            pl.BlockSpec((8, D), lambda b: (0, 0)),
        ],
        out_shape=[
            jax.ShapeDtypeStruct((PPAD, D), _f32),
            jax.ShapeDtypeStruct((8, D), _f32),
        ],
        scratch_shapes=[pltpu.VMEM((8, D), _f32)],
    )(agg1, W1, b1, pair_w)


def _layer2_body(hn_ref, hs_ref, m2_ref, st1_ref, g1_ref, be1_ref,
                 w2_ref, b2_ref, h2_ref, st2_ref, acc, *, bn):
    step = pl.program_id(0)

    @pl.when(step == 0)
    def _():
        acc[...] = jnp.zeros_like(acc)

    den = jnp.maximum(st1_ref[2:3], 1.0)
    mean = st1_ref[0:1] / den
    var = st1_ref[1:2] / den - mean * mean
    sc = g1_ref[...] * lax.rsqrt(var + EPS)
    sh = be1_ref[...] - mean * sc
    sc3 = sc.reshape(1, 1, D)
    sh3 = sh.reshape(1, 1, D)

    hn3 = hn_ref[...].reshape(bn, DEG, D)
    m2 = m2_ref[...]
    num = jax.nn.relu(hs_ref[...] * sc + sh)
    for k in range(DEG):
        h1k = jax.nn.relu(hn3[:, k, :] * sc3[0] + sh3[0])
        num = num + h1k * m2[:, k:k + 1]
    degs = jnp.sum(m2, axis=1, keepdims=True) + 1.0
    agg2 = num / degs
    h2p = jnp.dot(agg2, w2_ref[...], preferred_element_type=_f32) + b2_ref[...]
    h2_ref[...] = h2p

    rid = lax.broadcasted_iota(_i32, (bn, 1), 0) + step * bn
    w = (rid < N).astype(_f32)
    hw = h2p * w
    acc[0:1] = acc[0:1] + jnp.sum(hw, axis=0, keepdims=True)
    acc[1:2] = acc[1:2] + jnp.sum(h2p * hw, axis=0, keepdims=True)
    acc[2:3] = acc[2:3] + jnp.full((1, D), jnp.sum(w), _f32)
    st2_ref[...] = acc[...]


def _layer2(h1pre, mask2, stats1, gamma1, beta1, W2, b2):
    bn = 128
    grid = (NPAD // bn,)
    sb_blk = SB // bn
    return pl.pallas_call(
        functools.partial(_layer2_body, bn=bn),
        grid=grid,
        in_specs=[
            pl.BlockSpec((bn * DEG, D), lambda b: (b, 0)),
            pl.BlockSpec((bn, D), lambda b: (sb_blk + b, 0)),
            pl.BlockSpec((bn, DEG), lambda b: (b, 0)),
            pl.BlockSpec((8, D), lambda b: (0, 0)),
            pl.BlockSpec((1, D), lambda b: (0, 0)),
            pl.BlockSpec((1, D), lambda b: (0, 0)),
            pl.BlockSpec((D, D), lambda b: (0, 0)),
            pl.BlockSpec((1, D), lambda b: (0, 0)),
        ],
        out_specs=[
            pl.BlockSpec((bn, D), lambda b: (b, 0)),
            pl.BlockSpec((8, D), lambda b: (0, 0)),
        ],
        out_shape=[
            jax.ShapeDtypeStruct((NPAD, D), _f32),
            jax.ShapeDtypeStruct((8, D), _f32),
        ],
        scratch_shapes=[pltpu.VMEM((8, D), _f32)],
    )(h1pre, h1pre, mask2, stats1, gamma1, beta1, W2, b2)


def _final_body(h2_ref, st2_ref, g2_ref, be2_ref, out_ref):
    den = jnp.maximum(st2_ref[2:3], 1.0)
    mean = st2_ref[0:1] / den
    var = st2_ref[1:2] / den - mean * mean
    sc = g2_ref[...] * lax.rsqrt(var + EPS)
    sh = be2_ref[...] - mean * sc
    out_ref[...] = jax.nn.relu(h2_ref[...] * sc + sh)


def _final(h2pre, stats2, gamma2, beta2):
    bn = 512
    grid = (NPAD // bn,)
    return pl.pallas_call(
        _final_body,
        grid=grid,
        in_specs=[
            pl.BlockSpec((bn, D), lambda b: (b, 0)),
            pl.BlockSpec((8, D), lambda b: (0, 0)),
            pl.BlockSpec((1, D), lambda b: (0, 0)),
            pl.BlockSpec((1, D), lambda b: (0, 0)),
        ],
        out_specs=pl.BlockSpec((bn, D), lambda b: (b, 0)),
        out_shape=jax.ShapeDtypeStruct((NPAD, D), _f32),
    )(h2pre, stats2, gamma2, beta2)


@jax.jit
def _run(x, node_ts, nbr_ts, W1, b1, gamma1, beta1, W2, b2, gamma2, beta2,
         nbr_idx):
    xp = jnp.zeros((NPAD, D), _f32).at[:N].set(x)
    ntp = jnp.zeros((NPAD,), _f32).at[:N].set(node_ts)
    nbtp = jnp.full((NPAD, DEG), PADTS, _f32).at[:N].set(nbr_ts)
    nbip = jnp.zeros((NPAD, DEG), _i32).at[:N].set(nbr_idx.astype(_i32))
    mask2 = (nbtp <= ntp[:, None]).astype(_f32)
    wself = jnp.zeros((NPAD,), _f32).at[:N].set(1.0)
    zpad_i = jnp.zeros((PPAD - PREAL,), _i32)
    zpad_f = jnp.zeros((PPAD - PREAL,), _f32)
    pair_w = jnp.concatenate([mask2.reshape(-1), wself, zpad_f])

    tid_packed = jnp.concatenate(
        [lax.bitcast_convert_type(nbtp, _i32), nbip], axis=1)
    nbt_wide = jnp.pad(nbtp, ((0, 0), (0, D - DEG)))

    child, tsg = _sort_gather(tid_packed, xp, nbt_wide)
    table, idxn, idxs = _build_table(child, tsg, xp, nbtp,
                                     ntp.reshape(NPAD, 1), nbip)
    table_flat = table.reshape((DEG + 1) * NPAD, D)
    idx_flat = jnp.concatenate(
        [idxn.reshape(-1), idxs.reshape(-1), zpad_i])
    agg1 = _tab_gather(idx_flat, table_flat)
    h1pre, stats1 = _mm1(agg1, W1, b1.reshape(1, D), pair_w.reshape(PPAD, 1))
    h2pre, stats2 = _layer2(h1pre, mask2, stats1,
                            gamma1.reshape(1, D), beta1.reshape(1, D),
                            W2, b2.reshape(1, D))
    h2 = _final(h2pre, stats2, gamma2.reshape(1, D), beta2.reshape(1, D))
    return h2[:N]


def kernel(x, node_ts, nbr_ts, W1, b1, gamma1, beta1, W2, b2, gamma2, beta2,
           nbr_idx):
    return _run(x, node_ts, nbr_ts, W1, b1, gamma1, beta1, W2, b2,
                gamma2, beta2, nbr_idx)


# speedup vs baseline: 7.0811x; 1.0878x over previous
import functools

import jax
import jax.numpy as jnp
from jax import lax
from jax.experimental import pallas as pl
from jax.experimental.pallas import tpu as pltpu
from jax.experimental.pallas import tpu_sc as plsc

N = 10000
DEG = 16
D = 128

NW = 32
NPAD = 10240
CHN = 8
NCHA_TOT = (NPAD // 16) // CHN
NCHA_C0 = 56
NCHA_C1 = NCHA_TOT - NCHA_C0

SB = NPAD * DEG
PREAL = SB + NPAD
KC2 = 256
PPAD = 180224
CHC_TOT = (PPAD // 16) // KC2
CHC_C0 = 26
CHC_C1 = CHC_TOT - CHC_C0

PADTS = 3.0e38
EPS = 1e-5

_f32 = jnp.float32
_i32 = jnp.int32


def _sc_mesh():
    return plsc.VectorSubcoreMesh(core_axis_name="c", subcore_axis_name="s")


_SC_PARAMS = pltpu.CompilerParams(needs_layout_passes=False)


def _sga_sorts(tid, sidx, uidx):
    for n in range(CHN):
        ts = plsc.bitcast(tid[n, pl.ds(0, DEG)], _f32)
        ids = tid[n, pl.ds(DEG, DEG)]
        _, si = plsc.sort_key_val(ts, ids)
        sidx[pl.ds(n * DEG, DEG)] = si
        uidx[pl.ds(n * DEG, DEG)] = ids


def _sort_gather_body(tid_hbm, x_hbm, nbtw_hbm, child_hbm, tsg_hbm,
                      tid0, tid1, tid2, tid3, sx0, sx1, ux0, ux1,
                      rx0, rx1, rt0, rt1,
                      st0, st1, st2, st3, sgx0, sgx1, sgt0, sgt1,
                      swx0, swx1, swt0, swt1):
    core = lax.axis_index("c")
    base = (lax.axis_index("s") * (NPAD // 16)
            + jnp.where(core == 1, NCHA_C0 * CHN, 0))
    nch = jnp.where(core == 1, NCHA_C1, NCHA_C0)
    tid = [tid0, tid1, tid2, tid3]
    st = [st0, st1, st2, st3]
    sidx = [sx0, sx1]
    uidx = [ux0, ux1]
    rx = [rx0, rx1]
    rt = [rt0, rt1]
    sgx = [sgx0, sgx1]
    sgt = [sgt0, sgt1]
    swx = [swx0, swx1]
    swt = [swt0, swt1]
    NR = CHN * DEG

    def tid_copy(c, s):
        pltpu.async_copy(tid_hbm.at[pl.ds(base + c * CHN, CHN)], tid[s], st[s])

    def tid_wait(s):
        pltpu.make_async_copy(tid_hbm.at[pl.ds(0, CHN)], tid[s], st[s]).wait()

    def gathers(b):
        pltpu.async_copy(x_hbm.at[sidx[b]], rx[b], sgx[b])
        pltpu.async_copy(nbtw_hbm.at[uidx[b]], rt[b], sgt[b])

    def gathers_wait(b):
        pltpu.make_async_copy(x_hbm.at[pl.ds(0, NR)], rx[b], sgx[b]).wait()
        pltpu.make_async_copy(nbtw_hbm.at[pl.ds(0, NR)], rt[b], sgt[b]).wait()

    def writes_wait(b):
        pltpu.make_async_copy(rx[b], child_hbm.at[pl.ds(0, NR)], swx[b]).wait()
        pltpu.make_async_copy(rt[b], tsg_hbm.at[pl.ds(0, NR)], swt[b]).wait()

    tid_copy(0, 0)
    tid_copy(1, 1)
    tid_wait(0)
    _sga_sorts(tid0, sx0, ux0)
    gathers(0)

    def outer(g, carry):
        for j in range(4):
            c = g * 4 + j
            s1, s2 = (j + 1) % 4, (j + 2) % 4
            b, b1 = j % 2, (j + 1) % 2

            @pl.when(c + 2 < nch)
            def _():
                tid_copy(c + 2, s2)

            @pl.when(c + 1 < nch)
            def _():
                tid_wait(s1)
                _sga_sorts(tid[s1], sidx[b1], uidx[b1])

                @pl.when(c >= 1)
                def _():
                    writes_wait(b1)

                gathers(b1)

            @pl.when(c < nch)
            def _():
                gathers_wait(b)
                nb = base + c * CHN
                pltpu.async_copy(
                    rx[b], child_hbm.at[pl.ds(nb * DEG, NR)], swx[b])
                pltpu.async_copy(
                    rt[b], tsg_hbm.at[pl.ds(nb * DEG, NR)], swt[b])
        return carry

    lax.fori_loop(0, (NCHA_C0 + 3) // 4, outer, 0)
    for b in range(2):
        writes_wait(b)


def _sort_gather(tid_packed, xp, nbt_wide):
    f = pl.kernel(
        _sort_gather_body,
        out_type=(
            jax.ShapeDtypeStruct((NPAD * DEG, D), _f32),
            jax.ShapeDtypeStruct((NPAD * DEG, D), _f32),
        ),
        mesh=_sc_mesh(),
        scratch_types=(
            [pltpu.VMEM((CHN, 2 * DEG), _i32) for _ in range(4)]
            + [pltpu.VMEM((CHN * DEG,), _i32) for _ in range(4)]
            + [pltpu.VMEM((CHN * DEG, D), _f32) for _ in range(4)]
            + [pltpu.SemaphoreType.DMA for _ in range(12)]
        ),
        compiler_params=_SC_PARAMS,
    )
    return f(tid_packed, xp, nbt_wide)


def _table_body(child_ref, x_ref, tsg_ref, nbt_ref, nts_ref, nbi_ref,
                table_ref, idxn_ref, idxs_ref, *, bn):
    c3 = child_ref[...].reshape(bn, DEG, D)
    acc = x_ref[...]
    table_ref[0] = acc
    for k in range(1, DEG + 1):
        acc = acc + c3[:, k - 1, :]
        table_ref[k] = acc * (1.0 / (k + 1))
    tsg3 = tsg_ref[...].reshape(bn, DEG, D)[:, :, :DEG]
    t3 = nbt_ref[...][:, :, None]
    r = jnp.sum((tsg3 <= t3).astype(_i32), axis=2)
    idxn_ref[...] = r * NPAD + nbi_ref[...]
    rs = jnp.sum((nbt_ref[...] <= nts_ref[...]).astype(_i32), axis=1,
                 keepdims=True)
    nid = lax.broadcasted_iota(_i32, (bn, 1), 0) + pl.program_id(0) * bn
    idxs_ref[...] = rs * NPAD + nid


def _build_table(child, tsg, xp, nbtp, ntp_col, nbip):
    bn = 128
    grid = (NPAD // bn,)
    return pl.pallas_call(
        functools.partial(_table_body, bn=bn),
        grid=grid,
        in_specs=[
            pl.BlockSpec((bn * DEG, D), lambda b: (b, 0)),
            pl.BlockSpec((bn, D), lambda b: (b, 0)),
            pl.BlockSpec((bn * DEG, D), lambda b: (b, 0)),
            pl.BlockSpec((bn, DEG), lambda b: (b, 0)),
            pl.BlockSpec((bn, 1), lambda b: (b, 0)),
            pl.BlockSpec((bn, DEG), lambda b: (b, 0)),
        ],
        out_specs=[
            pl.BlockSpec((DEG + 1, bn, D), lambda b: (0, b, 0)),
            pl.BlockSpec((bn, DEG), lambda b: (b, 0)),
            pl.BlockSpec((bn, 1), lambda b: (b, 0)),
        ],
        out_shape=[
            jax.ShapeDtypeStruct((DEG + 1, NPAD, D), _f32),
            jax.ShapeDtypeStruct((NPAD, DEG), _i32),
            jax.ShapeDtypeStruct((NPAD, 1), _i32),
        ],
    )(child, xp, tsg, nbtp, ntp_col, nbip)


def _tab_gather_body(idx_hbm, table_hbm, agg1_hbm,
                     ix00, ix01, ix10, ix11, outr0, outr1,
                     si0, si1, sg0, sg1, sw0, sw1):
    core = lax.axis_index("c")
    base = (lax.axis_index("s") * (PPAD // 16)
            + jnp.where(core == 1, CHC_C0 * KC2, 0))
    nch = jnp.where(core == 1, CHC_C1, CHC_C0)
    ix = [(ix00, ix01), (ix10, ix11)]
    si = [si0, si1]
    outr = [outr0, outr1]
    sg = [sg0, sg1]
    sw = [sw0, sw1]

    def idx_copy(c, b):
        pb = base + c * KC2
        pltpu.async_copy(idx_hbm.at[pl.ds(pb, 128)], ix[b][0], si[b])
        pltpu.async_copy(idx_hbm.at[pl.ds(pb + 128, 128)], ix[b][1], si[b])

    def idx_wait(b):
        for h in range(2):
            pltpu.make_async_copy(
                idx_hbm.at[pl.ds(0, 128)], ix[b][h], si[b]).wait()

    def tgather(b):
        for h in range(2):
            pltpu.async_copy(
                table_hbm.at[ix[b][h]], outr[b].at[pl.ds(h * 128, 128)],
                sg[b])

    def tgather_wait(b):
        for h in range(2):
            pltpu.make_async_copy(
                table_hbm.at[pl.ds(0, 128)], outr[b].at[pl.ds(h * 128, 128)],
                sg[b]).wait()

    idx_copy(0, 0)
    idx_copy(1, 1)
    idx_wait(0)
    tgather(0)

    def outer(g, carry):
        for j in range(2):
            c = g * 2 + j
            b, b1 = j, (j + 1) % 2

            @pl.when(c + 1 < nch)
            def _():
                idx_wait(b1)

                @pl.when(c >= 1)
                def _():
                    pltpu.make_async_copy(
                        outr[b1], agg1_hbm.at[pl.ds(0, KC2)], sw[b1]).wait()

                tgather(b1)

            @pl.when(c < nch)
            def _():
                tgather_wait(b)
                pltpu.async_copy(
                    outr[b], agg1_hbm.at[pl.ds(base + c * KC2, KC2)], sw[b])

            @pl.when(c + 2 < nch)
            def _():
                idx_copy(c + 2, b)
        return carry

    lax.fori_loop(0, (CHC_C0 + 1) // 2, outer, 0)
    for b in range(2):
        pltpu.make_async_copy(
            outr[b], agg1_hbm.at[pl.ds(0, KC2)], sw[b]).wait()


def _tab_gather(idx_flat, table_flat):
    f = pl.kernel(
        _tab_gather_body,
        out_type=jax.ShapeDtypeStruct((PPAD, D), _f32),
        mesh=_sc_mesh(),
        scratch_types=(
            [pltpu.VMEM((128,), _i32) for _ in range(4)]
            + [pltpu.VMEM((KC2, D), _f32) for _ in range(2)]
            + [pltpu.SemaphoreType.DMA for _ in range(6)]
        ),
        compiler_params=_SC_PARAMS,
    )
    return f(idx_flat, table_flat)


def _mm1_body(agg_ref, w1_ref, b1_ref, pw_ref, h_ref, st_ref, acc):
    step = pl.program_id(0)

    @pl.when(step == 0)
    def _():
        acc[...] = jnp.zeros_like(acc)

    a = agg_ref[...]
    h = jnp.dot(a, w1_ref[...], preferred_element_type=_f32) + b1_ref[...]
    h_ref[...] = h
    w = pw_ref[...]
    hw = h * w
    s1 = jnp.sum(hw, axis=0, keepdims=True)
    s2 = jnp.sum(h * hw, axis=0, keepdims=True)
    d = jnp.sum(w)
    acc[0:1] = acc[0:1] + s1
    acc[1:2] = acc[1:2] + s2
    acc[2:3] = acc[2:3] + jnp.full((1, D), d, _f32)
    st_ref[...] = acc[...]


def _mm1(agg1, W1, b1, pair_w):
    bp = 2048
    grid = (PPAD // bp,)
    return pl.pallas_call(
        _mm1_body,
        grid=grid,
        in_specs=[
            pl.BlockSpec((bp, D), lambda b: (b, 0)),
            pl.BlockSpec((D, D), lambda b: (0, 0)),
            pl.BlockSpec((1, D), lambda b: (0, 0)),
            pl.BlockSpec((bp, 1), lambda b: (b, 0)),
        ],
        out_specs=[
            pl.BlockSpec((bp, D), lambda b: (b, 0)),
            pl.BlockSpec((8, D), lambda b: (0, 0)),
        ],
        out_shape=[
            jax.ShapeDtypeStruct((PPAD, D), _f32),
            jax.ShapeDtypeStruct((8, D), _f32),
        ],
        scratch_shapes=[pltpu.VMEM((8, D), _f32)],
    )(agg1, W1, b1, pair_w)


def _layer2_body(hn_ref, hs_ref, m2_ref, st1_ref, g1_ref, be1_ref,
                 w2_ref, b2_ref, h2_ref, st2_ref, acc, *, bn):
    step = pl.program_id(0)

    @pl.when(step == 0)
    def _():
        acc[...] = jnp.zeros_like(acc)

    den = jnp.maximum(st1_ref[2:3], 1.0)
    mean = st1_ref[0:1] / den
    var = st1_ref[1:2] / den - mean * mean
    sc = g1_ref[...] * lax.rsqrt(var + EPS)
    sh = be1_ref[...] - mean * sc
    sc3 = sc.reshape(1, 1, D)
    sh3 = sh.reshape(1, 1, D)

    hn3 = hn_ref[...].reshape(bn, DEG, D)
    m2 = m2_ref[...]
    num = jax.nn.relu(hs_ref[...] * sc + sh)
    for k in range(DEG):
        h1k = jax.nn.relu(hn3[:, k, :] * sc3[0] + sh3[0])
        num = num + h1k * m2[:, k:k + 1]
    degs = jnp.sum(m2, axis=1, keepdims=True) + 1.0
    agg2 = num / degs
    h2p = jnp.dot(agg2, w2_ref[...], preferred_element_type=_f32) + b2_ref[...]
    h2_ref[...] = h2p

    rid = lax.broadcasted_iota(_i32, (bn, 1), 0) + step * bn
    w = (rid < N).astype(_f32)
    hw = h2p * w
    acc[0:1] = acc[0:1] + jnp.sum(hw, axis=0, keepdims=True)
    acc[1:2] = acc[1:2] + jnp.sum(h2p * hw, axis=0, keepdims=True)
    acc[2:3] = acc[2:3] + jnp.full((1, D), jnp.sum(w), _f32)
    st2_ref[...] = acc[...]


def _layer2(h1pre, mask2, stats1, gamma1, beta1, W2, b2):
    bn = 256
    grid = (NPAD // bn,)
    sb_blk = SB // bn
    return pl.pallas_call(
        functools.partial(_layer2_body, bn=bn),
        grid=grid,
        in_specs=[
            pl.BlockSpec((bn * DEG, D), lambda b: (b, 0)),
            pl.BlockSpec((bn, D), lambda b: (sb_blk + b, 0)),
            pl.BlockSpec((bn, DEG), lambda b: (b, 0)),
            pl.BlockSpec((8, D), lambda b: (0, 0)),
            pl.BlockSpec((1, D), lambda b: (0, 0)),
            pl.BlockSpec((1, D), lambda b: (0, 0)),
            pl.BlockSpec((D, D), lambda b: (0, 0)),
            pl.BlockSpec((1, D), lambda b: (0, 0)),
        ],
        out_specs=[
            pl.BlockSpec((bn, D), lambda b: (b, 0)),
            pl.BlockSpec((8, D), lambda b: (0, 0)),
        ],
        out_shape=[
            jax.ShapeDtypeStruct((NPAD, D), _f32),
            jax.ShapeDtypeStruct((8, D), _f32),
        ],
        scratch_shapes=[pltpu.VMEM((8, D), _f32)],
    )(h1pre, h1pre, mask2, stats1, gamma1, beta1, W2, b2)


def _final_body(h2_ref, st2_ref, g2_ref, be2_ref, out_ref):
    den = jnp.maximum(st2_ref[2:3], 1.0)
    mean = st2_ref[0:1] / den
    var = st2_ref[1:2] / den - mean * mean
    sc = g2_ref[...] * lax.rsqrt(var + EPS)
    sh = be2_ref[...] - mean * sc
    out_ref[...] = jax.nn.relu(h2_ref[...] * sc + sh)


def _final(h2pre, stats2, gamma2, beta2):
    bn = 512
    grid = (NPAD // bn,)
    return pl.pallas_call(
        _final_body,
        grid=grid,
        in_specs=[
            pl.BlockSpec((bn, D), lambda b: (b, 0)),
            pl.BlockSpec((8, D), lambda b: (0, 0)),
            pl.BlockSpec((1, D), lambda b: (0, 0)),
            pl.BlockSpec((1, D), lambda b: (0, 0)),
        ],
        out_specs=pl.BlockSpec((bn, D), lambda b: (b, 0)),
        out_shape=jax.ShapeDtypeStruct((NPAD, D), _f32),
    )(h2pre, stats2, gamma2, beta2)


@jax.jit
def _run(x, node_ts, nbr_ts, W1, b1, gamma1, beta1, W2, b2, gamma2, beta2,
         nbr_idx):
    xp = jnp.zeros((NPAD, D), _f32).at[:N].set(x)
    ntp = jnp.zeros((NPAD,), _f32).at[:N].set(node_ts)
    nbtp = jnp.full((NPAD, DEG), PADTS, _f32).at[:N].set(nbr_ts)
    nbip = jnp.zeros((NPAD, DEG), _i32).at[:N].set(nbr_idx.astype(_i32))
    mask2 = (nbtp <= ntp[:, None]).astype(_f32)
    wself = jnp.zeros((NPAD,), _f32).at[:N].set(1.0)
    zpad_i = jnp.zeros((PPAD - PREAL,), _i32)
    zpad_f = jnp.zeros((PPAD - PREAL,), _f32)
    pair_w = jnp.concatenate([mask2.reshape(-1), wself, zpad_f])

    tid_packed = jnp.concatenate(
        [lax.bitcast_convert_type(nbtp, _i32), nbip], axis=1)
    nbt_wide = jnp.pad(nbtp, ((0, 0), (0, D - DEG)))

    child, tsg = _sort_gather(tid_packed, xp, nbt_wide)
    table, idxn, idxs = _build_table(child, tsg, xp, nbtp,
                                     ntp.reshape(NPAD, 1), nbip)
    table_flat = table.reshape((DEG + 1) * NPAD, D)
    idx_flat = jnp.concatenate(
        [idxn.reshape(-1), idxs.reshape(-1), zpad_i])
    agg1 = _tab_gather(idx_flat, table_flat)
    h1pre, stats1 = _mm1(agg1, W1, b1.reshape(1, D), pair_w.reshape(PPAD, 1))
    h2pre, stats2 = _layer2(h1pre, mask2, stats1,
                            gamma1.reshape(1, D), beta1.reshape(1, D),
                            W2, b2.reshape(1, D))
    h2 = _final(h2pre, stats2, gamma2.reshape(1, D), beta2.reshape(1, D))
    return h2[:N]


def kernel(x, node_ts, nbr_ts, W1, b1, gamma1, beta1, W2, b2, gamma2, beta2,
           nbr_idx):
    return _run(x, node_ts, nbr_ts, W1, b1, gamma1, beta1, W2, b2,
                gamma2, beta2, nbr_idx)


# speedup vs baseline: 7.1321x; 1.0072x over previous
import functools

import jax
import jax.numpy as jnp
from jax import lax
from jax.experimental import pallas as pl
from jax.experimental.pallas import tpu as pltpu
from jax.experimental.pallas import tpu_sc as plsc

N = 10000
DEG = 16
D = 128

NW = 32
NPAD = 10240
CHN = 8
NCHA_TOT = (NPAD // 16) // CHN
NCHA_C0 = 56
NCHA_C1 = NCHA_TOT - NCHA_C0

SB = NPAD * DEG
PREAL = SB + NPAD
KC2 = 256
PPAD = 180224
CHC_TOT = (PPAD // 16) // KC2
CHC_C0 = 26
CHC_C1 = CHC_TOT - CHC_C0

PADTS = 3.0e38
EPS = 1e-5

_f32 = jnp.float32
_i32 = jnp.int32


def _sc_mesh():
    return plsc.VectorSubcoreMesh(core_axis_name="c", subcore_axis_name="s")


_SC_PARAMS = pltpu.CompilerParams(needs_layout_passes=False)


def _sga_sorts(tid, sidx, uidx):
    for n in range(CHN):
        ts = plsc.bitcast(tid[n, pl.ds(0, DEG)], _f32)
        ids = tid[n, pl.ds(DEG, DEG)]
        _, si = plsc.sort_key_val(ts, ids)
        sidx[pl.ds(n * DEG, DEG)] = si
        uidx[pl.ds(n * DEG, DEG)] = ids


def _sort_gather_body(tid_hbm, x_hbm, nbtw_hbm, child_hbm, tsg_hbm,
                      tid0, tid1, tid2, tid3, sx0, sx1, ux0, ux1,
                      rx0, rx1, rt0, rt1,
                      st0, st1, st2, st3, sgx0, sgx1, sgt0, sgt1,
                      swx0, swx1, swt0, swt1):
    core = lax.axis_index("c")
    base = (lax.axis_index("s") * (NPAD // 16)
            + jnp.where(core == 1, NCHA_C0 * CHN, 0))
    nch = jnp.where(core == 1, NCHA_C1, NCHA_C0)
    tid = [tid0, tid1, tid2, tid3]
    st = [st0, st1, st2, st3]
    sidx = [sx0, sx1]
    uidx = [ux0, ux1]
    rx = [rx0, rx1]
    rt = [rt0, rt1]
    sgx = [sgx0, sgx1]
    sgt = [sgt0, sgt1]
    swx = [swx0, swx1]
    swt = [swt0, swt1]
    NR = CHN * DEG

    def tid_copy(c, s):
        pltpu.async_copy(tid_hbm.at[pl.ds(base + c * CHN, CHN)], tid[s], st[s])

    def tid_wait(s):
        pltpu.make_async_copy(tid_hbm.at[pl.ds(0, CHN)], tid[s], st[s]).wait()

    def gathers(b):
        pltpu.async_copy(x_hbm.at[sidx[b]], rx[b], sgx[b])
        pltpu.async_copy(nbtw_hbm.at[uidx[b]], rt[b], sgt[b])

    def gathers_wait(b):
        pltpu.make_async_copy(x_hbm.at[pl.ds(0, NR)], rx[b], sgx[b]).wait()
        pltpu.make_async_copy(nbtw_hbm.at[pl.ds(0, NR)], rt[b], sgt[b]).wait()

    def writes_wait(b):
        pltpu.make_async_copy(rx[b], child_hbm.at[pl.ds(0, NR)], swx[b]).wait()
        pltpu.make_async_copy(rt[b], tsg_hbm.at[pl.ds(0, NR)], swt[b]).wait()

    tid_copy(0, 0)
    tid_copy(1, 1)
    tid_wait(0)
    _sga_sorts(tid0, sx0, ux0)
    gathers(0)

    def outer(g, carry):
        for j in range(4):
            c = g * 4 + j
            s1, s2 = (j + 1) % 4, (j + 2) % 4
            b, b1 = j % 2, (j + 1) % 2

            @pl.when(c + 2 < nch)
            def _():
                tid_copy(c + 2, s2)

            @pl.when(c + 1 < nch)
            def _():
                tid_wait(s1)
                _sga_sorts(tid[s1], sidx[b1], uidx[b1])

                @pl.when(c >= 1)
                def _():
                    writes_wait(b1)

                gathers(b1)

            @pl.when(c < nch)
            def _():
                gathers_wait(b)
                nb = base + c * CHN
                pltpu.async_copy(
                    rx[b], child_hbm.at[pl.ds(nb * DEG, NR)], swx[b])
                pltpu.async_copy(
                    rt[b], tsg_hbm.at[pl.ds(nb * DEG, NR)], swt[b])
        return carry

    lax.fori_loop(0, (NCHA_C0 + 3) // 4, outer, 0)
    for b in range(2):
        writes_wait(b)


def _sort_gather(tid_packed, xp, nbt_wide):
    f = pl.kernel(
        _sort_gather_body,
        out_type=(
            jax.ShapeDtypeStruct((NPAD * DEG, D), _f32),
            jax.ShapeDtypeStruct((NPAD * DEG, D), _f32),
        ),
        mesh=_sc_mesh(),
        scratch_types=(
            [pltpu.VMEM((CHN, 2 * DEG), _i32) for _ in range(4)]
            + [pltpu.VMEM((CHN * DEG,), _i32) for _ in range(4)]
            + [pltpu.VMEM((CHN * DEG, D), _f32) for _ in range(4)]
            + [pltpu.SemaphoreType.DMA for _ in range(12)]
        ),
        compiler_params=_SC_PARAMS,
    )
    return f(tid_packed, xp, nbt_wide)


def _table_body(child_ref, x_ref, tsg_ref, nbt_ref, nts_ref, nbi_ref,
                table_ref, idxn_ref, idxs_ref, *, bn):
    c3 = child_ref[...].reshape(bn, DEG, D)
    acc = x_ref[...]
    table_ref[0] = acc
    for k in range(1, DEG + 1):
        acc = acc + c3[:, k - 1, :]
        table_ref[k] = acc * (1.0 / (k + 1))
    tsg3 = tsg_ref[...].reshape(bn, DEG, D)[:, :, :DEG]
    t3 = nbt_ref[...][:, :, None]
    r = jnp.sum((tsg3 <= t3).astype(_i32), axis=2)
    idxn_ref[...] = r * NPAD + nbi_ref[...]
    rs = jnp.sum((nbt_ref[...] <= nts_ref[...]).astype(_i32), axis=1,
                 keepdims=True)
    nid = lax.broadcasted_iota(_i32, (bn, 1), 0) + pl.program_id(0) * bn
    idxs_ref[...] = rs * NPAD + nid


def _build_table(child, tsg, xp, nbtp, ntp_col, nbip):
    bn = 128
    grid = (NPAD // bn,)
    return pl.pallas_call(
        functools.partial(_table_body, bn=bn),
        grid=grid,
        in_specs=[
            pl.BlockSpec((bn * DEG, D), lambda b: (b, 0)),
            pl.BlockSpec((bn, D), lambda b: (b, 0)),
            pl.BlockSpec((bn * DEG, D), lambda b: (b, 0)),
            pl.BlockSpec((bn, DEG), lambda b: (b, 0)),
            pl.BlockSpec((bn, 1), lambda b: (b, 0)),
            pl.BlockSpec((bn, DEG), lambda b: (b, 0)),
        ],
        out_specs=[
            pl.BlockSpec((DEG + 1, bn, D), lambda b: (0, b, 0)),
            pl.BlockSpec((bn, DEG), lambda b: (b, 0)),
            pl.BlockSpec((bn, 1), lambda b: (b, 0)),
        ],
        out_shape=[
            jax.ShapeDtypeStruct((DEG + 1, NPAD, D), _f32),
            jax.ShapeDtypeStruct((NPAD, DEG), _i32),
            jax.ShapeDtypeStruct((NPAD, 1), _i32),
        ],
    )(child, xp, tsg, nbtp, ntp_col, nbip)


def _tab_gather_body(idx_hbm, table_hbm, agg1_hbm,
                     ix00, ix01, ix10, ix11, outr0, outr1,
                     si0, si1, sg0, sg1, sw0, sw1):
    core = lax.axis_index("c")
    base = (lax.axis_index("s") * (PPAD // 16)
            + jnp.where(core == 1, CHC_C0 * KC2, 0))
    nch = jnp.where(core == 1, CHC_C1, CHC_C0)
    ix = [(ix00, ix01), (ix10, ix11)]
    si = [si0, si1]
    outr = [outr0, outr1]
    sg = [sg0, sg1]
    sw = [sw0, sw1]

    def idx_copy(c, b):
        pb = base + c * KC2
        pltpu.async_copy(idx_hbm.at[pl.ds(pb, 128)], ix[b][0], si[b])
        pltpu.async_copy(idx_hbm.at[pl.ds(pb + 128, 128)], ix[b][1], si[b])

    def idx_wait(b):
        for h in range(2):
            pltpu.make_async_copy(
                idx_hbm.at[pl.ds(0, 128)], ix[b][h], si[b]).wait()

    def tgather(b):
        for h in range(2):
            pltpu.async_copy(
                table_hbm.at[ix[b][h]], outr[b].at[pl.ds(h * 128, 128)],
                sg[b])

    def tgather_wait(b):
        for h in range(2):
            pltpu.make_async_copy(
                table_hbm.at[pl.ds(0, 128)], outr[b].at[pl.ds(h * 128, 128)],
                sg[b]).wait()

    idx_copy(0, 0)
    idx_copy(1, 1)
    idx_wait(0)
    tgather(0)

    def outer(g, carry):
        for j in range(2):
            c = g * 2 + j
            b, b1 = j, (j + 1) % 2

            @pl.when(c + 1 < nch)
            def _():
                idx_wait(b1)

                @pl.when(c >= 1)
                def _():
                    pltpu.make_async_copy(
                        outr[b1], agg1_hbm.at[pl.ds(0, KC2)], sw[b1]).wait()

                tgather(b1)

            @pl.when(c < nch)
            def _():
                tgather_wait(b)
                pltpu.async_copy(
                    outr[b], agg1_hbm.at[pl.ds(base + c * KC2, KC2)], sw[b])

            @pl.when(c + 2 < nch)
            def _():
                idx_copy(c + 2, b)
        return carry

    lax.fori_loop(0, (CHC_C0 + 1) // 2, outer, 0)
    for b in range(2):
        pltpu.make_async_copy(
            outr[b], agg1_hbm.at[pl.ds(0, KC2)], sw[b]).wait()


def _tab_gather(idx_flat, table_flat):
    f = pl.kernel(
        _tab_gather_body,
        out_type=jax.ShapeDtypeStruct((PPAD, D), _f32),
        mesh=_sc_mesh(),
        scratch_types=(
            [pltpu.VMEM((128,), _i32) for _ in range(4)]
            + [pltpu.VMEM((KC2, D), _f32) for _ in range(2)]
            + [pltpu.SemaphoreType.DMA for _ in range(6)]
        ),
        compiler_params=_SC_PARAMS,
    )
    return f(idx_flat, table_flat)


def _mm1_body(agg_ref, w1_ref, b1_ref, pw_ref, h_ref, st_ref, acc):
    step = pl.program_id(0)

    @pl.when(step == 0)
    def _():
        acc[...] = jnp.zeros_like(acc)

    a = agg_ref[...]
    h = jnp.dot(a, w1_ref[...], preferred_element_type=_f32) + b1_ref[...]
    h_ref[...] = h
    w = pw_ref[...]
    hw = h * w
    s1 = jnp.sum(hw, axis=0, keepdims=True)
    s2 = jnp.sum(h * hw, axis=0, keepdims=True)
    d = jnp.sum(w)
    acc[0:1] = acc[0:1] + s1
    acc[1:2] = acc[1:2] + s2
    acc[2:3] = acc[2:3] + jnp.full((1, D), d, _f32)
    st_ref[...] = acc[...]


def _mm1(agg1, W1, b1, pair_w):
    bp = 4096
    grid = (PPAD // bp,)
    return pl.pallas_call(
        _mm1_body,
        grid=grid,
        in_specs=[
            pl.BlockSpec((bp, D), lambda b: (b, 0)),
            pl.BlockSpec((D, D), lambda b: (0, 0)),
            pl.BlockSpec((1, D), lambda b: (0, 0)),
            pl.BlockSpec((bp, 1), lambda b: (b, 0)),
        ],
        out_specs=[
            pl.BlockSpec((bp, D), lambda b: (b, 0)),
            pl.BlockSpec((8, D), lambda b: (0, 0)),
        ],
        out_shape=[
            jax.ShapeDtypeStruct((PPAD, D), _f32),
            jax.ShapeDtypeStruct((8, D), _f32),
        ],
        scratch_shapes=[pltpu.VMEM((8, D), _f32)],
    )(agg1, W1, b1, pair_w)


def _layer2_body(hn_ref, hs_ref, wn_ref, st1_ref, g1_ref, be1_ref,
                 w2_ref, b2_ref, h2_ref, st2_ref, acc, *, bn):
    step = pl.program_id(0)

    @pl.when(step == 0)
    def _():
        acc[...] = jnp.zeros_like(acc)

    den = jnp.maximum(st1_ref[2:3], 1.0)
    mean = st1_ref[0:1] / den
    var = st1_ref[1:2] / den - mean * mean
    sc = g1_ref[...] * lax.rsqrt(var + EPS)
    sh = be1_ref[...] - mean * sc
    sc3 = sc.reshape(1, 1, D)
    sh3 = sh.reshape(1, 1, D)

    hn3 = hn_ref[...].reshape(bn, DEG, D)
    wn3 = wn_ref[...].reshape(bn, DEG, 1)
    num = jax.nn.relu(hs_ref[...] * sc + sh)
    degs = jnp.ones((bn, 1), _f32)
    for k in range(DEG):
        h1k = jax.nn.relu(hn3[:, k, :] * sc3[0] + sh3[0])
        num = num + h1k * wn3[:, k, :]
        degs = degs + wn3[:, k, :]
    agg2 = num / degs
    h2p = jnp.dot(agg2, w2_ref[...], preferred_element_type=_f32) + b2_ref[...]
    h2_ref[...] = h2p

    rid = lax.broadcasted_iota(_i32, (bn, 1), 0) + step * bn
    w = (rid < N).astype(_f32)
    hw = h2p * w
    acc[0:1] = acc[0:1] + jnp.sum(hw, axis=0, keepdims=True)
    acc[1:2] = acc[1:2] + jnp.sum(h2p * hw, axis=0, keepdims=True)
    acc[2:3] = acc[2:3] + jnp.full((1, D), jnp.sum(w), _f32)
    st2_ref[...] = acc[...]


def _layer2(h1pre, wnbr, stats1, gamma1, beta1, W2, b2):
    bn = 256
    grid = (NPAD // bn,)
    sb_blk = SB // bn
    return pl.pallas_call(
        functools.partial(_layer2_body, bn=bn),
        grid=grid,
        in_specs=[
            pl.BlockSpec((bn * DEG, D), lambda b: (b, 0)),
            pl.BlockSpec((bn, D), lambda b: (sb_blk + b, 0)),
            pl.BlockSpec((bn * DEG, 1), lambda b: (b, 0)),
            pl.BlockSpec((8, D), lambda b: (0, 0)),
            pl.BlockSpec((1, D), lambda b: (0, 0)),
            pl.BlockSpec((1, D), lambda b: (0, 0)),
            pl.BlockSpec((D, D), lambda b: (0, 0)),
            pl.BlockSpec((1, D), lambda b: (0, 0)),
        ],
        out_specs=[
            pl.BlockSpec((bn, D), lambda b: (b, 0)),
            pl.BlockSpec((8, D), lambda b: (0, 0)),
        ],
        out_shape=[
            jax.ShapeDtypeStruct((NPAD, D), _f32),
            jax.ShapeDtypeStruct((8, D), _f32),
        ],
        scratch_shapes=[pltpu.VMEM((8, D), _f32)],
    )(h1pre, h1pre, wnbr, stats1, gamma1, beta1, W2, b2)


def _final_body(h2_ref, st2_ref, g2_ref, be2_ref, out_ref):
    den = jnp.maximum(st2_ref[2:3], 1.0)
    mean = st2_ref[0:1] / den
    var = st2_ref[1:2] / den - mean * mean
    sc = g2_ref[...] * lax.rsqrt(var + EPS)
    sh = be2_ref[...] - mean * sc
    out_ref[...] = jax.nn.relu(h2_ref[...] * sc + sh)


def _final(h2pre, stats2, gamma2, beta2):
    bn = 512
    grid = (NPAD // bn,)
    return pl.pallas_call(
        _final_body,
        grid=grid,
        in_specs=[
            pl.BlockSpec((bn, D), lambda b: (b, 0)),
            pl.BlockSpec((8, D), lambda b: (0, 0)),
            pl.BlockSpec((1, D), lambda b: (0, 0)),
            pl.BlockSpec((1, D), lambda b: (0, 0)),
        ],
        out_specs=pl.BlockSpec((bn, D), lambda b: (b, 0)),
        out_shape=jax.ShapeDtypeStruct((NPAD, D), _f32),
    )(h2pre, stats2, gamma2, beta2)


@jax.jit
def _run(x, node_ts, nbr_ts, W1, b1, gamma1, beta1, W2, b2, gamma2, beta2,
         nbr_idx):
    xp = jnp.zeros((NPAD, D), _f32).at[:N].set(x)
    ntp = jnp.zeros((NPAD,), _f32).at[:N].set(node_ts)
    nbtp = jnp.full((NPAD, DEG), PADTS, _f32).at[:N].set(nbr_ts)
    nbip = jnp.zeros((NPAD, DEG), _i32).at[:N].set(nbr_idx.astype(_i32))
    mask2 = (nbtp <= ntp[:, None]).astype(_f32)
    wself = jnp.zeros((NPAD,), _f32).at[:N].set(1.0)
    zpad_i = jnp.zeros((PPAD - PREAL,), _i32)
    zpad_f = jnp.zeros((PPAD - PREAL,), _f32)
    pair_w = jnp.concatenate([mask2.reshape(-1), wself, zpad_f])

    tid_packed = jnp.concatenate(
        [lax.bitcast_convert_type(nbtp, _i32), nbip], axis=1)
    nbt_wide = jnp.pad(nbtp, ((0, 0), (0, D - DEG)))

    child, tsg = _sort_gather(tid_packed, xp, nbt_wide)
    table, idxn, idxs = _build_table(child, tsg, xp, nbtp,
                                     ntp.reshape(NPAD, 1), nbip)
    table_flat = table.reshape((DEG + 1) * NPAD, D)
    idx_flat = jnp.concatenate(
        [idxn.reshape(-1), idxs.reshape(-1), zpad_i])
    agg1 = _tab_gather(idx_flat, table_flat)
    h1pre, stats1 = _mm1(agg1, W1, b1.reshape(1, D), pair_w.reshape(PPAD, 1))
    h2pre, stats2 = _layer2(h1pre, mask2.reshape(SB, 1), stats1,
                            gamma1.reshape(1, D), beta1.reshape(1, D),
                            W2, b2.reshape(1, D))
    h2 = _final(h2pre, stats2, gamma2.reshape(1, D), beta2.reshape(1, D))
    return h2[:N]


def kernel(x, node_ts, nbr_ts, W1, b1, gamma1, beta1, W2, b2, gamma2, beta2,
           nbr_idx):
    return _run(x, node_ts, nbr_ts, W1, b1, gamma1, beta1, W2, b2,
                gamma2, beta2, nbr_idx)


# speedup vs baseline: 7.5989x; 1.0654x over previous
import functools

import jax
import jax.numpy as jnp
from jax import lax
from jax.experimental import pallas as pl
from jax.experimental.pallas import tpu as pltpu
from jax.experimental.pallas import tpu_sc as plsc

N = 10000
DEG = 16
D = 128

NW = 32
NPAD = 10240
CHN = 8
NHALF = NPAD // 2
NCHA_TOT = (NHALF // 16) // CHN
NCHA_C0 = 28
NCHA_C1 = NCHA_TOT - NCHA_C0

SB = NPAD * DEG
PREAL = SB + NPAD
KC2 = 256
PPAD = 180224
CHC_TOT = (PPAD // 16) // KC2
CHC_C0 = 26
CHC_C1 = CHC_TOT - CHC_C0

PADTS = 3.0e38
EPS = 1e-5

_f32 = jnp.float32
_i32 = jnp.int32


def _sc_mesh():
    return plsc.VectorSubcoreMesh(core_axis_name="c", subcore_axis_name="s")


_SC_PARAMS = pltpu.CompilerParams(needs_layout_passes=False)


def _sga_sorts(tid, sidx, uidx):
    for n in range(CHN):
        ts = plsc.bitcast(tid[n, pl.ds(0, DEG)], _f32)
        ids = tid[n, pl.ds(DEG, DEG)]
        _, si = plsc.sort_key_val(ts, ids)
        sidx[pl.ds(n * DEG, DEG)] = si
        uidx[pl.ds(n * DEG, DEG)] = ids


def _sort_gather_body(tid_hbm, x_hbm, nbtw_hbm, child_hbm, tsg_hbm,
                      tid0, tid1, tid2, tid3, sx0, sx1, ux0, ux1,
                      rx0, rx1, rt0, rt1,
                      st0, st1, st2, st3, sgx0, sgx1, sgt0, sgt1,
                      swx0, swx1, swt0, swt1):
    core = lax.axis_index("c")
    base = (lax.axis_index("s") * (NHALF // 16)
            + jnp.where(core == 1, NCHA_C0 * CHN, 0))
    nch = jnp.where(core == 1, NCHA_C1, NCHA_C0)
    tid = [tid0, tid1, tid2, tid3]
    st = [st0, st1, st2, st3]
    sidx = [sx0, sx1]
    uidx = [ux0, ux1]
    rx = [rx0, rx1]
    rt = [rt0, rt1]
    sgx = [sgx0, sgx1]
    sgt = [sgt0, sgt1]
    swx = [swx0, swx1]
    swt = [swt0, swt1]
    NR = CHN * DEG

    def tid_copy(c, s):
        pltpu.async_copy(tid_hbm.at[pl.ds(base + c * CHN, CHN)], tid[s], st[s])

    def tid_wait(s):
        pltpu.make_async_copy(tid_hbm.at[pl.ds(0, CHN)], tid[s], st[s]).wait()

    def gathers(b):
        pltpu.async_copy(x_hbm.at[sidx[b]], rx[b], sgx[b])
        pltpu.async_copy(nbtw_hbm.at[uidx[b]], rt[b], sgt[b])

    def gathers_wait(b):
        pltpu.make_async_copy(x_hbm.at[pl.ds(0, NR)], rx[b], sgx[b]).wait()
        pltpu.make_async_copy(nbtw_hbm.at[pl.ds(0, NR)], rt[b], sgt[b]).wait()

    def writes_wait(b):
        pltpu.make_async_copy(rx[b], child_hbm.at[pl.ds(0, NR)], swx[b]).wait()
        pltpu.make_async_copy(rt[b], tsg_hbm.at[pl.ds(0, NR)], swt[b]).wait()

    tid_copy(0, 0)
    tid_copy(1, 1)
    tid_wait(0)
    _sga_sorts(tid0, sx0, ux0)
    gathers(0)

    def outer(g, carry):
        for j in range(4):
            c = g * 4 + j
            s1, s2 = (j + 1) % 4, (j + 2) % 4
            b, b1 = j % 2, (j + 1) % 2

            @pl.when(c + 2 < nch)
            def _():
                tid_copy(c + 2, s2)

            @pl.when(c + 1 < nch)
            def _():
                tid_wait(s1)
                _sga_sorts(tid[s1], sidx[b1], uidx[b1])

                @pl.when(c >= 1)
                def _():
                    writes_wait(b1)

                gathers(b1)

            @pl.when(c < nch)
            def _():
                gathers_wait(b)
                nb = base + c * CHN
                pltpu.async_copy(
                    rx[b], child_hbm.at[pl.ds(nb * DEG, NR)], swx[b])
                pltpu.async_copy(
                    rt[b], tsg_hbm.at[pl.ds(nb * DEG, NR)], swt[b])
        return carry

    lax.fori_loop(0, (NCHA_C0 + 3) // 4, outer, 0)
    for b in range(2):
        writes_wait(b)


def _sort_gather(tid_half, xp, nbt_wide):
    f = pl.kernel(
        _sort_gather_body,
        out_type=(
            jax.ShapeDtypeStruct((NHALF * DEG, D), _f32),
            jax.ShapeDtypeStruct((NHALF * DEG, D), _f32),
        ),
        mesh=_sc_mesh(),
        scratch_types=(
            [pltpu.VMEM((CHN, 2 * DEG), _i32) for _ in range(4)]
            + [pltpu.VMEM((CHN * DEG,), _i32) for _ in range(4)]
            + [pltpu.VMEM((CHN * DEG, D), _f32) for _ in range(4)]
            + [pltpu.SemaphoreType.DMA for _ in range(12)]
        ),
        compiler_params=_SC_PARAMS,
    )
    return f(tid_half, xp, nbt_wide)


def _table_body(child_ref, x_ref, tsg_ref, nbt_ref, nts_ref, nbi_ref,
                table_ref, idxn_ref, idxs_ref, *, bn, noff):
    c3 = child_ref[...].reshape(bn, DEG, D)
    acc = x_ref[...]
    table_ref[0] = acc
    for k in range(1, DEG + 1):
        acc = acc + c3[:, k - 1, :]
        table_ref[k] = acc * (1.0 / (k + 1))
    tsg3 = tsg_ref[...].reshape(bn, DEG, D)[:, :, :DEG]
    t3 = nbt_ref[...][:, :, None]
    r = jnp.sum((tsg3 <= t3).astype(_i32), axis=2)
    idxn_ref[...] = r * NPAD + nbi_ref[...]
    rs = jnp.sum((nbt_ref[...] <= nts_ref[...]).astype(_i32), axis=1,
                 keepdims=True)
    nid = (lax.broadcasted_iota(_i32, (bn, 1), 0)
           + pl.program_id(0) * bn + noff)
    idxs_ref[...] = rs * NPAD + nid


def _build_table(child, tsg, xp, nbtp, ntp_col, nbip, half, prev=None):
    bn = 128
    grid = (NHALF // bn,)
    off = half * (NHALF // bn)

    def body(child_ref, x_ref, tsg_ref, nbt_ref, nts_ref, nbi_ref,
             *rest):
        _table_body(child_ref, x_ref, tsg_ref, nbt_ref, nts_ref, nbi_ref,
                    rest[-3], rest[-2], rest[-1], bn=bn, noff=half * NHALF)

    in_specs = [
        pl.BlockSpec((bn * DEG, D), lambda b: (b, 0)),
        pl.BlockSpec((bn, D), lambda b: (off + b, 0)),
        pl.BlockSpec((bn * DEG, D), lambda b: (b, 0)),
        pl.BlockSpec((bn, DEG), lambda b: (off + b, 0)),
        pl.BlockSpec((bn, 1), lambda b: (off + b, 0)),
        pl.BlockSpec((bn, DEG), lambda b: (off + b, 0)),
    ]
    args = [child, xp, tsg, nbtp, ntp_col, nbip]
    aliases = {}
    if prev is not None:
        in_specs += [pl.BlockSpec(memory_space=pl.ANY)] * 3
        args += list(prev)
        aliases = {6: 0, 7: 1, 8: 2}
    return pl.pallas_call(
        body,
        grid=grid,
        in_specs=in_specs,
        out_specs=[
            pl.BlockSpec((DEG + 1, bn, D), lambda b: (0, off + b, 0)),
            pl.BlockSpec((bn, DEG), lambda b: (off + b, 0)),
            pl.BlockSpec((bn, 1), lambda b: (off + b, 0)),
        ],
        out_shape=[
            jax.ShapeDtypeStruct((DEG + 1, NPAD, D), _f32),
            jax.ShapeDtypeStruct((NPAD, DEG), _i32),
            jax.ShapeDtypeStruct((NPAD, 1), _i32),
        ],
        input_output_aliases=aliases,
    )(*args)


def _tab_gather_body(idx_hbm, table_hbm, agg1_hbm,
                     ix00, ix01, ix10, ix11, outr0, outr1,
                     si0, si1, sg0, sg1, sw0, sw1):
    core = lax.axis_index("c")
    base = (lax.axis_index("s") * (PPAD // 16)
            + jnp.where(core == 1, CHC_C0 * KC2, 0))
    nch = jnp.where(core == 1, CHC_C1, CHC_C0)
    ix = [(ix00, ix01), (ix10, ix11)]
    si = [si0, si1]
    outr = [outr0, outr1]
    sg = [sg0, sg1]
    sw = [sw0, sw1]

    def idx_copy(c, b):
        pb = base + c * KC2
        pltpu.async_copy(idx_hbm.at[pl.ds(pb, 128)], ix[b][0], si[b])
        pltpu.async_copy(idx_hbm.at[pl.ds(pb + 128, 128)], ix[b][1], si[b])

    def idx_wait(b):
        for h in range(2):
            pltpu.make_async_copy(
                idx_hbm.at[pl.ds(0, 128)], ix[b][h], si[b]).wait()

    def tgather(b):
        for h in range(2):
            pltpu.async_copy(
                table_hbm.at[ix[b][h]], outr[b].at[pl.ds(h * 128, 128)],
                sg[b])

    def tgather_wait(b):
        for h in range(2):
            pltpu.make_async_copy(
                table_hbm.at[pl.ds(0, 128)], outr[b].at[pl.ds(h * 128, 128)],
                sg[b]).wait()

    idx_copy(0, 0)
    idx_copy(1, 1)
    idx_wait(0)
    tgather(0)

    def outer(g, carry):
        for j in range(2):
            c = g * 2 + j
            b, b1 = j, (j + 1) % 2

            @pl.when(c + 1 < nch)
            def _():
                idx_wait(b1)

                @pl.when(c >= 1)
                def _():
                    pltpu.make_async_copy(
                        outr[b1], agg1_hbm.at[pl.ds(0, KC2)], sw[b1]).wait()

                tgather(b1)

            @pl.when(c < nch)
            def _():
                tgather_wait(b)
                pltpu.async_copy(
                    outr[b], agg1_hbm.at[pl.ds(base + c * KC2, KC2)], sw[b])

            @pl.when(c + 2 < nch)
            def _():
                idx_copy(c + 2, b)
        return carry

    lax.fori_loop(0, (CHC_C0 + 1) // 2, outer, 0)
    for b in range(2):
        pltpu.make_async_copy(
            outr[b], agg1_hbm.at[pl.ds(0, KC2)], sw[b]).wait()


def _tab_gather(idx_flat, table_flat):
    f = pl.kernel(
        _tab_gather_body,
        out_type=jax.ShapeDtypeStruct((PPAD, D), _f32),
        mesh=_sc_mesh(),
        scratch_types=(
            [pltpu.VMEM((128,), _i32) for _ in range(4)]
            + [pltpu.VMEM((KC2, D), _f32) for _ in range(2)]
            + [pltpu.SemaphoreType.DMA for _ in range(6)]
        ),
        compiler_params=_SC_PARAMS,
    )
    return f(idx_flat, table_flat)


def _mm1_body(agg_ref, w1_ref, b1_ref, pw_ref, h_ref, st_ref, acc):
    step = pl.program_id(0)

    @pl.when(step == 0)
    def _():
        acc[...] = jnp.zeros_like(acc)

    a = agg_ref[...]
    h = jnp.dot(a, w1_ref[...], preferred_element_type=_f32) + b1_ref[...]
    h_ref[...] = h
    w = pw_ref[...]
    hw = h * w
    s1 = jnp.sum(hw, axis=0, keepdims=True)
    s2 = jnp.sum(h * hw, axis=0, keepdims=True)
    d = jnp.sum(w)
    acc[0:1] = acc[0:1] + s1
    acc[1:2] = acc[1:2] + s2
    acc[2:3] = acc[2:3] + jnp.full((1, D), d, _f32)
    st_ref[...] = acc[...]


def _mm1(agg1, W1, b1, pair_w):
    bp = 4096
    grid = (PPAD // bp,)
    return pl.pallas_call(
        _mm1_body,
        grid=grid,
        in_specs=[
            pl.BlockSpec((bp, D), lambda b: (b, 0)),
            pl.BlockSpec((D, D), lambda b: (0, 0)),
            pl.BlockSpec((1, D), lambda b: (0, 0)),
            pl.BlockSpec((bp, 1), lambda b: (b, 0)),
        ],
        out_specs=[
            pl.BlockSpec((bp, D), lambda b: (b, 0)),
            pl.BlockSpec((8, D), lambda b: (0, 0)),
        ],
        out_shape=[
            jax.ShapeDtypeStruct((PPAD, D), _f32),
            jax.ShapeDtypeStruct((8, D), _f32),
        ],
        scratch_shapes=[pltpu.VMEM((8, D), _f32)],
    )(agg1, W1, b1, pair_w)


def _layer2_body(hn_ref, hs_ref, wn_ref, st1_ref, g1_ref, be1_ref,
                 w2_ref, b2_ref, h2_ref, st2_ref, acc, *, bn):
    step = pl.program_id(0)

    @pl.when(step == 0)
    def _():
        acc[...] = jnp.zeros_like(acc)

    den = jnp.maximum(st1_ref[2:3], 1.0)
    mean = st1_ref[0:1] / den
    var = st1_ref[1:2] / den - mean * mean
    sc = g1_ref[...] * lax.rsqrt(var + EPS)
    sh = be1_ref[...] - mean * sc
    sc3 = sc.reshape(1, 1, D)
    sh3 = sh.reshape(1, 1, D)

    hn3 = hn_ref[...].reshape(bn, DEG, D)
    wn3 = wn_ref[...].reshape(bn, DEG, 1)
    num = jax.nn.relu(hs_ref[...] * sc + sh)
    degs = jnp.ones((bn, 1), _f32)
    for k in range(DEG):
        h1k = jax.nn.relu(hn3[:, k, :] * sc3[0] + sh3[0])
        num = num + h1k * wn3[:, k, :]
        degs = degs + wn3[:, k, :]
    agg2 = num / degs
    h2p = jnp.dot(agg2, w2_ref[...], preferred_element_type=_f32) + b2_ref[...]
    h2_ref[...] = h2p

    rid = lax.broadcasted_iota(_i32, (bn, 1), 0) + step * bn
    w = (rid < N).astype(_f32)
    hw = h2p * w
    acc[0:1] = acc[0:1] + jnp.sum(hw, axis=0, keepdims=True)
    acc[1:2] = acc[1:2] + jnp.sum(h2p * hw, axis=0, keepdims=True)
    acc[2:3] = acc[2:3] + jnp.full((1, D), jnp.sum(w), _f32)
    st2_ref[...] = acc[...]


def _layer2(h1pre, wnbr, stats1, gamma1, beta1, W2, b2):
    bn = 256
    grid = (NPAD // bn,)
    sb_blk = SB // bn
    return pl.pallas_call(
        functools.partial(_layer2_body, bn=bn),
        grid=grid,
        in_specs=[
            pl.BlockSpec((bn * DEG, D), lambda b: (b, 0)),
            pl.BlockSpec((bn, D), lambda b: (sb_blk + b, 0)),
            pl.BlockSpec((bn * DEG, 1), lambda b: (b, 0)),
            pl.BlockSpec((8, D), lambda b: (0, 0)),
            pl.BlockSpec((1, D), lambda b: (0, 0)),
            pl.BlockSpec((1, D), lambda b: (0, 0)),
            pl.BlockSpec((D, D), lambda b: (0, 0)),
            pl.BlockSpec((1, D), lambda b: (0, 0)),
        ],
        out_specs=[
            pl.BlockSpec((bn, D), lambda b: (b, 0)),
            pl.BlockSpec((8, D), lambda b: (0, 0)),
        ],
        out_shape=[
            jax.ShapeDtypeStruct((NPAD, D), _f32),
            jax.ShapeDtypeStruct((8, D), _f32),
        ],
        scratch_shapes=[pltpu.VMEM((8, D), _f32)],
    )(h1pre, h1pre, wnbr, stats1, gamma1, beta1, W2, b2)


def _final_body(h2_ref, st2_ref, g2_ref, be2_ref, out_ref):
    den = jnp.maximum(st2_ref[2:3], 1.0)
    mean = st2_ref[0:1] / den
    var = st2_ref[1:2] / den - mean * mean
    sc = g2_ref[...] * lax.rsqrt(var + EPS)
    sh = be2_ref[...] - mean * sc
    out_ref[...] = jax.nn.relu(h2_ref[...] * sc + sh)


def _final(h2pre, stats2, gamma2, beta2):
    bn = 512
    grid = (NPAD // bn,)
    return pl.pallas_call(
        _final_body,
        grid=grid,
        in_specs=[
            pl.BlockSpec((bn, D), lambda b: (b, 0)),
            pl.BlockSpec((8, D), lambda b: (0, 0)),
            pl.BlockSpec((1, D), lambda b: (0, 0)),
            pl.BlockSpec((1, D), lambda b: (0, 0)),
        ],
        out_specs=pl.BlockSpec((bn, D), lambda b: (b, 0)),
        out_shape=jax.ShapeDtypeStruct((NPAD, D), _f32),
    )(h2pre, stats2, gamma2, beta2)


@jax.jit
def _run(x, node_ts, nbr_ts, W1, b1, gamma1, beta1, W2, b2, gamma2, beta2,
         nbr_idx):
    xp = jnp.zeros((NPAD, D), _f32).at[:N].set(x)
    ntp = jnp.zeros((NPAD,), _f32).at[:N].set(node_ts)
    nbtp = jnp.full((NPAD, DEG), PADTS, _f32).at[:N].set(nbr_ts)
    nbip = jnp.zeros((NPAD, DEG), _i32).at[:N].set(nbr_idx.astype(_i32))
    mask2 = (nbtp <= ntp[:, None]).astype(_f32)
    wself = jnp.zeros((NPAD,), _f32).at[:N].set(1.0)
    zpad_i = jnp.zeros((PPAD - PREAL,), _i32)
    zpad_f = jnp.zeros((PPAD - PREAL,), _f32)
    pair_w = jnp.concatenate([mask2.reshape(-1), wself, zpad_f])

    tid_packed = jnp.concatenate(
        [lax.bitcast_convert_type(nbtp, _i32), nbip], axis=1)
    nbt_wide = jnp.pad(nbtp, ((0, 0), (0, D - DEG)))

    ntp_col = ntp.reshape(NPAD, 1)
    child1, tsg1 = _sort_gather(tid_packed[:NHALF], xp, nbt_wide)
    child2, tsg2 = _sort_gather(tid_packed[NHALF:], xp, nbt_wide)
    prev = _build_table(child1, tsg1, xp, nbtp, ntp_col, nbip, 0)
    table, idxn, idxs = _build_table(child2, tsg2, xp, nbtp, ntp_col,
                                     nbip, 1, prev)
    table_flat = table.reshape((DEG + 1) * NPAD, D)
    idx_flat = jnp.concatenate(
        [idxn.reshape(-1), idxs.reshape(-1), zpad_i])
    agg1 = _tab_gather(idx_flat, table_flat)
    h1pre, stats1 = _mm1(agg1, W1, b1.reshape(1, D), pair_w.reshape(PPAD, 1))
    h2pre, stats2 = _layer2(h1pre, mask2.reshape(SB, 1), stats1,
                            gamma1.reshape(1, D), beta1.reshape(1, D),
                            W2, b2.reshape(1, D))
    h2 = _final(h2pre, stats2, gamma2.reshape(1, D), beta2.reshape(1, D))
    return h2[:N]


def kernel(x, node_ts, nbr_ts, W1, b1, gamma1, beta1, W2, b2, gamma2, beta2,
           nbr_idx):
    return _run(x, node_ts, nbr_ts, W1, b1, gamma1, beta1, W2, b2,
                gamma2, beta2, nbr_idx)


# speedup vs baseline: 7.8470x; 1.0327x over previous
import functools

import jax
import jax.numpy as jnp
from jax import lax
from jax.experimental import pallas as pl
from jax.experimental.pallas import tpu as pltpu
from jax.experimental.pallas import tpu_sc as plsc

N = 10000
DEG = 16
D = 128

NW = 32
NPAD = 10240
CHN = 8
NHALF = NPAD // 2
NCHA_TOT = (NHALF // 16) // CHN
NCHA_C0 = 24
NCHA_C1 = NCHA_TOT - NCHA_C0

SB = NPAD * DEG
PREAL = SB + NPAD
KC2 = 256
PPAD = 180224
CHC_TOT = (PPAD // 16) // KC2
CHC_C0 = 26
CHC_C1 = CHC_TOT - CHC_C0

PADTS = 3.0e38
EPS = 1e-5

_f32 = jnp.float32
_i32 = jnp.int32


def _sc_mesh():
    return plsc.VectorSubcoreMesh(core_axis_name="c", subcore_axis_name="s")


_SC_PARAMS = pltpu.CompilerParams(needs_layout_passes=False)


def _sga_sorts(tid, sidx, uidx):
    for n in range(CHN):
        ts = plsc.bitcast(tid[n, pl.ds(0, DEG)], _f32)
        ids = tid[n, pl.ds(DEG, DEG)]
        _, si = plsc.sort_key_val(ts, ids)
        sidx[pl.ds(n * DEG, DEG)] = si
        uidx[pl.ds(n * DEG, DEG)] = ids


def _sort_gather_body(tid_hbm, x_hbm, nbtw_hbm, child_hbm, tsg_hbm,
                      tid0, tid1, tid2, tid3, sx0, sx1, ux0, ux1,
                      rx0, rx1, rt0, rt1,
                      st0, st1, st2, st3, sgx0, sgx1, sgt0, sgt1,
                      swx0, swx1, swt0, swt1):
    core = lax.axis_index("c")
    base = (lax.axis_index("s") * (NHALF // 16)
            + jnp.where(core == 1, NCHA_C0 * CHN, 0))
    nch = jnp.where(core == 1, NCHA_C1, NCHA_C0)
    tid = [tid0, tid1, tid2, tid3]
    st = [st0, st1, st2, st3]
    sidx = [sx0, sx1]
    uidx = [ux0, ux1]
    rx = [rx0, rx1]
    rt = [rt0, rt1]
    sgx = [sgx0, sgx1]
    sgt = [sgt0, sgt1]
    swx = [swx0, swx1]
    swt = [swt0, swt1]
    NR = CHN * DEG

    def tid_copy(c, s):
        pltpu.async_copy(tid_hbm.at[pl.ds(base + c * CHN, CHN)], tid[s], st[s])

    def tid_wait(s):
        pltpu.make_async_copy(tid_hbm.at[pl.ds(0, CHN)], tid[s], st[s]).wait()

    def gathers(b):
        pltpu.async_copy(x_hbm.at[sidx[b]], rx[b], sgx[b])
        pltpu.async_copy(nbtw_hbm.at[uidx[b]], rt[b], sgt[b])

    def gathers_wait(b):
        pltpu.make_async_copy(x_hbm.at[pl.ds(0, NR)], rx[b], sgx[b]).wait()
        pltpu.make_async_copy(nbtw_hbm.at[pl.ds(0, NR)], rt[b], sgt[b]).wait()

    def writes_wait(b):
        pltpu.make_async_copy(rx[b], child_hbm.at[pl.ds(0, NR)], swx[b]).wait()
        pltpu.make_async_copy(rt[b], tsg_hbm.at[pl.ds(0, NR)], swt[b]).wait()

    tid_copy(0, 0)
    tid_copy(1, 1)
    tid_wait(0)
    _sga_sorts(tid0, sx0, ux0)
    gathers(0)

    def outer(g, carry):
        for j in range(4):
            c = g * 4 + j
            s1, s2 = (j + 1) % 4, (j + 2) % 4
            b, b1 = j % 2, (j + 1) % 2

            @pl.when(c + 2 < nch)
            def _():
                tid_copy(c + 2, s2)

            @pl.when(c + 1 < nch)
            def _():
                tid_wait(s1)
                _sga_sorts(tid[s1], sidx[b1], uidx[b1])

                @pl.when(c >= 1)
                def _():
                    writes_wait(b1)

                gathers(b1)

            @pl.when(c < nch)
            def _():
                gathers_wait(b)
                nb = base + c * CHN
                pltpu.async_copy(
                    rx[b], child_hbm.at[pl.ds(nb * DEG, NR)], swx[b])
                pltpu.async_copy(
                    rt[b], tsg_hbm.at[pl.ds(nb * DEG, NR)], swt[b])
        return carry

    lax.fori_loop(0, (NCHA_C0 + 3) // 4, outer, 0)
    for b in range(2):
        writes_wait(b)


def _sort_gather(tid_half, xp, nbt_wide):
    f = pl.kernel(
        _sort_gather_body,
        out_type=(
            jax.ShapeDtypeStruct((NHALF * DEG, D), _f32),
            jax.ShapeDtypeStruct((NHALF * DEG, D), _f32),
        ),
        mesh=_sc_mesh(),
        scratch_types=(
            [pltpu.VMEM((CHN, 2 * DEG), _i32) for _ in range(4)]
            + [pltpu.VMEM((CHN * DEG,), _i32) for _ in range(4)]
            + [pltpu.VMEM((CHN * DEG, D), _f32) for _ in range(4)]
            + [pltpu.SemaphoreType.DMA for _ in range(12)]
        ),
        compiler_params=_SC_PARAMS,
    )
    return f(tid_half, xp, nbt_wide)


def _table_body(child_ref, x_ref, tsg_ref, nbt_ref, nts_ref, nbi_ref,
                table_ref, idxn_ref, idxs_ref, *, bn, noff):
    c3 = child_ref[...].reshape(bn, DEG, D)
    acc = x_ref[...]
    table_ref[0] = acc
    for k in range(1, DEG + 1):
        acc = acc + c3[:, k - 1, :]
        table_ref[k] = acc * (1.0 / (k + 1))
    tsg3 = tsg_ref[...].reshape(bn, DEG, D)[:, :, :DEG]
    t3 = nbt_ref[...][:, :, None]
    r = jnp.sum((tsg3 <= t3).astype(_i32), axis=2)
    idxn_ref[...] = r * NPAD + nbi_ref[...]
    rs = jnp.sum((nbt_ref[...] <= nts_ref[...]).astype(_i32), axis=1,
                 keepdims=True)
    nid = (lax.broadcasted_iota(_i32, (bn, 1), 0)
           + pl.program_id(0) * bn + noff)
    idxs_ref[...] = rs * NPAD + nid


def _build_table(child, tsg, xp, nbtp, ntp_col, nbip, half, prev=None):
    bn = 128
    grid = (NHALF // bn,)
    off = half * (NHALF // bn)

    def body(child_ref, x_ref, tsg_ref, nbt_ref, nts_ref, nbi_ref,
             *rest):
        _table_body(child_ref, x_ref, tsg_ref, nbt_ref, nts_ref, nbi_ref,
                    rest[-3], rest[-2], rest[-1], bn=bn, noff=half * NHALF)

    in_specs = [
        pl.BlockSpec((bn * DEG, D), lambda b: (b, 0)),
        pl.BlockSpec((bn, D), lambda b: (off + b, 0)),
        pl.BlockSpec((bn * DEG, D), lambda b: (b, 0)),
        pl.BlockSpec((bn, DEG), lambda b: (off + b, 0)),
        pl.BlockSpec((bn, 1), lambda b: (off + b, 0)),
        pl.BlockSpec((bn, DEG), lambda b: (off + b, 0)),
    ]
    args = [child, xp, tsg, nbtp, ntp_col, nbip]
    aliases = {}
    if prev is not None:
        in_specs += [pl.BlockSpec(memory_space=pl.ANY)] * 3
        args += list(prev)
        aliases = {6: 0, 7: 1, 8: 2}
    return pl.pallas_call(
        body,
        grid=grid,
        in_specs=in_specs,
        out_specs=[
            pl.BlockSpec((DEG + 1, bn, D), lambda b: (0, off + b, 0)),
            pl.BlockSpec((bn, DEG), lambda b: (off + b, 0)),
            pl.BlockSpec((bn, 1), lambda b: (off + b, 0)),
        ],
        out_shape=[
            jax.ShapeDtypeStruct((DEG + 1, NPAD, D), _f32),
            jax.ShapeDtypeStruct((NPAD, DEG), _i32),
            jax.ShapeDtypeStruct((NPAD, 1), _i32),
        ],
        input_output_aliases=aliases,
    )(*args)


def _tab_gather_body(idx_hbm, table_hbm, agg1_hbm,
                     ix00, ix01, ix10, ix11, outr0, outr1,
                     si0, si1, sg0, sg1, sw0, sw1):
    core = lax.axis_index("c")
    base = (lax.axis_index("s") * (PPAD // 16)
            + jnp.where(core == 1, CHC_C0 * KC2, 0))
    nch = jnp.where(core == 1, CHC_C1, CHC_C0)
    ix = [(ix00, ix01), (ix10, ix11)]
    si = [si0, si1]
    outr = [outr0, outr1]
    sg = [sg0, sg1]
    sw = [sw0, sw1]

    def idx_copy(c, b):
        pb = base + c * KC2
        pltpu.async_copy(idx_hbm.at[pl.ds(pb, 128)], ix[b][0], si[b])
        pltpu.async_copy(idx_hbm.at[pl.ds(pb + 128, 128)], ix[b][1], si[b])

    def idx_wait(b):
        for h in range(2):
            pltpu.make_async_copy(
                idx_hbm.at[pl.ds(0, 128)], ix[b][h], si[b]).wait()

    def tgather(b):
        for h in range(2):
            pltpu.async_copy(
                table_hbm.at[ix[b][h]], outr[b].at[pl.ds(h * 128, 128)],
                sg[b])

    def tgather_wait(b):
        for h in range(2):
            pltpu.make_async_copy(
                table_hbm.at[pl.ds(0, 128)], outr[b].at[pl.ds(h * 128, 128)],
                sg[b]).wait()

    idx_copy(0, 0)
    idx_copy(1, 1)
    idx_wait(0)
    tgather(0)

    def outer(g, carry):
        for j in range(2):
            c = g * 2 + j
            b, b1 = j, (j + 1) % 2

            @pl.when(c + 1 < nch)
            def _():
                idx_wait(b1)

                @pl.when(c >= 1)
                def _():
                    pltpu.make_async_copy(
                        outr[b1], agg1_hbm.at[pl.ds(0, KC2)], sw[b1]).wait()

                tgather(b1)

            @pl.when(c < nch)
            def _():
                tgather_wait(b)
                pltpu.async_copy(
                    outr[b], agg1_hbm.at[pl.ds(base + c * KC2, KC2)], sw[b])

            @pl.when(c + 2 < nch)
            def _():
                idx_copy(c + 2, b)
        return carry

    lax.fori_loop(0, (CHC_C0 + 1) // 2, outer, 0)
    for b in range(2):
        pltpu.make_async_copy(
            outr[b], agg1_hbm.at[pl.ds(0, KC2)], sw[b]).wait()


def _tab_gather(idx_flat, table_flat):
    f = pl.kernel(
        _tab_gather_body,
        out_type=jax.ShapeDtypeStruct((PPAD, D), _f32),
        mesh=_sc_mesh(),
        scratch_types=(
            [pltpu.VMEM((128,), _i32) for _ in range(4)]
            + [pltpu.VMEM((KC2, D), _f32) for _ in range(2)]
            + [pltpu.SemaphoreType.DMA for _ in range(6)]
        ),
        compiler_params=_SC_PARAMS,
    )
    return f(idx_flat, table_flat)


def _mm1_body(agg_ref, w1_ref, b1_ref, pw_ref, h_ref, st_ref, acc):
    step = pl.program_id(0)

    @pl.when(step == 0)
    def _():
        acc[...] = jnp.zeros_like(acc)

    a = agg_ref[...]
    h = jnp.dot(a, w1_ref[...], preferred_element_type=_f32) + b1_ref[...]
    h_ref[...] = h
    w = pw_ref[...]
    hw = h * w
    s1 = jnp.sum(hw, axis=0, keepdims=True)
    s2 = jnp.sum(h * hw, axis=0, keepdims=True)
    d = jnp.sum(w)
    acc[0:1] = acc[0:1] + s1
    acc[1:2] = acc[1:2] + s2
    acc[2:3] = acc[2:3] + jnp.full((1, D), d, _f32)
    st_ref[...] = acc[...]


def _mm1(agg1, W1, b1, pair_w):
    bp = 4096
    grid = (PPAD // bp,)
    return pl.pallas_call(
        _mm1_body,
        grid=grid,
        in_specs=[
            pl.BlockSpec((bp, D), lambda b: (b, 0)),
            pl.BlockSpec((D, D), lambda b: (0, 0)),
            pl.BlockSpec((1, D), lambda b: (0, 0)),
            pl.BlockSpec((bp, 1), lambda b: (b, 0)),
        ],
        out_specs=[
            pl.BlockSpec((bp, D), lambda b: (b, 0)),
            pl.BlockSpec((8, D), lambda b: (0, 0)),
        ],
        out_shape=[
            jax.ShapeDtypeStruct((PPAD, D), _f32),
            jax.ShapeDtypeStruct((8, D), _f32),
        ],
        scratch_shapes=[pltpu.VMEM((8, D), _f32)],
    )(agg1, W1, b1, pair_w)


def _layer2_body(hn_ref, hs_ref, wn_ref, m2_ref, st1_ref, g1_ref, be1_ref,
                 w2_ref, b2_ref, h2_ref, st2_ref, acc, scr, *, bn):
    step = pl.program_id(0)

    @pl.when(step == 0)
    def _():
        acc[...] = jnp.zeros_like(acc)

    den = jnp.maximum(st1_ref[2:3], 1.0)
    mean = st1_ref[0:1] / den
    var = st1_ref[1:2] / den - mean * mean
    sc = g1_ref[...] * lax.rsqrt(var + EPS)
    sh = be1_ref[...] - mean * sc

    hw = jax.nn.relu(hn_ref[...] * sc + sh) * wn_ref[...]
    scr[...] = hw.reshape(bn, DEG, D)
    num = jax.nn.relu(hs_ref[...] * sc + sh)
    for k in range(DEG):
        num = num + scr[:, k, :]
    degs = jnp.sum(m2_ref[...], axis=1, keepdims=True) + 1.0
    agg2 = num / degs
    h2p = jnp.dot(agg2, w2_ref[...], preferred_element_type=_f32) + b2_ref[...]
    h2_ref[...] = h2p

    rid = lax.broadcasted_iota(_i32, (bn, 1), 0) + step * bn
    w = (rid < N).astype(_f32)
    hw = h2p * w
    acc[0:1] = acc[0:1] + jnp.sum(hw, axis=0, keepdims=True)
    acc[1:2] = acc[1:2] + jnp.sum(h2p * hw, axis=0, keepdims=True)
    acc[2:3] = acc[2:3] + jnp.full((1, D), jnp.sum(w), _f32)
    st2_ref[...] = acc[...]


def _layer2(h1pre, wnbr, mask2, stats1, gamma1, beta1, W2, b2):
    bn = 256
    grid = (NPAD // bn,)
    sb_blk = SB // bn
    return pl.pallas_call(
        functools.partial(_layer2_body, bn=bn),
        grid=grid,
        in_specs=[
            pl.BlockSpec((bn * DEG, D), lambda b: (b, 0)),
            pl.BlockSpec((bn, D), lambda b: (sb_blk + b, 0)),
            pl.BlockSpec((bn * DEG, 1), lambda b: (b, 0)),
            pl.BlockSpec((bn, DEG), lambda b: (b, 0)),
            pl.BlockSpec((8, D), lambda b: (0, 0)),
            pl.BlockSpec((1, D), lambda b: (0, 0)),
            pl.BlockSpec((1, D), lambda b: (0, 0)),
            pl.BlockSpec((D, D), lambda b: (0, 0)),
            pl.BlockSpec((1, D), lambda b: (0, 0)),
        ],
        out_specs=[
            pl.BlockSpec((bn, D), lambda b: (b, 0)),
            pl.BlockSpec((8, D), lambda b: (0, 0)),
        ],
        out_shape=[
            jax.ShapeDtypeStruct((NPAD, D), _f32),
            jax.ShapeDtypeStruct((8, D), _f32),
        ],
        scratch_shapes=[pltpu.VMEM((8, D), _f32),
                        pltpu.VMEM((bn, DEG, D), _f32)],
    )(h1pre, h1pre, wnbr, mask2, stats1, gamma1, beta1, W2, b2)


def _final_body(h2_ref, st2_ref, g2_ref, be2_ref, out_ref):
    den = jnp.maximum(st2_ref[2:3], 1.0)
    mean = st2_ref[0:1] / den
    var = st2_ref[1:2] / den - mean * mean
    sc = g2_ref[...] * lax.rsqrt(var + EPS)
    sh = be2_ref[...] - mean * sc
    out_ref[...] = jax.nn.relu(h2_ref[...] * sc + sh)


def _final(h2pre, stats2, gamma2, beta2):
    bn = 512
    grid = (NPAD // bn,)
    return pl.pallas_call(
        _final_body,
        grid=grid,
        in_specs=[
            pl.BlockSpec((bn, D), lambda b: (b, 0)),
            pl.BlockSpec((8, D), lambda b: (0, 0)),
            pl.BlockSpec((1, D), lambda b: (0, 0)),
            pl.BlockSpec((1, D), lambda b: (0, 0)),
        ],
        out_specs=pl.BlockSpec((bn, D), lambda b: (b, 0)),
        out_shape=jax.ShapeDtypeStruct((NPAD, D), _f32),
    )(h2pre, stats2, gamma2, beta2)


@jax.jit
def _run(x, node_ts, nbr_ts, W1, b1, gamma1, beta1, W2, b2, gamma2, beta2,
         nbr_idx):
    xp = jnp.zeros((NPAD, D), _f32).at[:N].set(x)
    ntp = jnp.zeros((NPAD,), _f32).at[:N].set(node_ts)
    nbtp = jnp.full((NPAD, DEG), PADTS, _f32).at[:N].set(nbr_ts)
    nbip = jnp.zeros((NPAD, DEG), _i32).at[:N].set(nbr_idx.astype(_i32))
    mask2 = (nbtp <= ntp[:, None]).astype(_f32)
    wself = jnp.zeros((NPAD,), _f32).at[:N].set(1.0)
    zpad_i = jnp.zeros((PPAD - PREAL,), _i32)
    zpad_f = jnp.zeros((PPAD - PREAL,), _f32)
    pair_w = jnp.concatenate([mask2.reshape(-1), wself, zpad_f])

    tid_packed = jnp.concatenate(
        [lax.bitcast_convert_type(nbtp, _i32), nbip], axis=1)
    nbt_wide = jnp.pad(nbtp, ((0, 0), (0, D - DEG)))

    ntp_col = ntp.reshape(NPAD, 1)
    child1, tsg1 = _sort_gather(tid_packed[:NHALF], xp, nbt_wide)
    child2, tsg2 = _sort_gather(tid_packed[NHALF:], xp, nbt_wide)
    prev = _build_table(child1, tsg1, xp, nbtp, ntp_col, nbip, 0)
    table, idxn, idxs = _build_table(child2, tsg2, xp, nbtp, ntp_col,
                                     nbip, 1, prev)
    table_flat = table.reshape((DEG + 1) * NPAD, D)
    idx_flat = jnp.concatenate(
        [idxn.reshape(-1), idxs.reshape(-1), zpad_i])
    agg1 = _tab_gather(idx_flat, table_flat)
    h1pre, stats1 = _mm1(agg1, W1, b1.reshape(1, D), pair_w.reshape(PPAD, 1))
    h2pre, stats2 = _layer2(h1pre, mask2.reshape(SB, 1), mask2, stats1,
                            gamma1.reshape(1, D), beta1.reshape(1, D),
                            W2, b2.reshape(1, D))
    h2 = _final(h2pre, stats2, gamma2.reshape(1, D), beta2.reshape(1, D))
    return h2[:N]


def kernel(x, node_ts, nbr_ts, W1, b1, gamma1, beta1, W2, b2, gamma2, beta2,
           nbr_idx):
    return _run(x, node_ts, nbr_ts, W1, b1, gamma1, beta1, W2, b2,
                gamma2, beta2, nbr_idx)
